# baseline (speedup 1.0000x reference)
import functools

import jax
import jax.numpy as jnp
from jax import lax
from jax.experimental import pallas as pl
from jax.experimental.pallas import tpu as pltpu
from jax.experimental.pallas import tpu_sc as plsc

N_NODES = 10000
N_EDGES = 160000
DIM = 256
HALF = 128
NUM_BASIS = 16
CUTOFF = 5.0
N_GRAPHS = 8

NC = 2
NS = 16
NW = NC * NS
NP = 10240
EP = 163840
CHUNK = 128
E_PER_W = EP // NW
N_CHUNKS = E_PER_W // CHUNK
N_PER_W = NP // NW
ACC_ROWS = NP

_f32 = jnp.float32
_i32 = jnp.int32
_bf16 = jnp.bfloat16


def _silu(x):
    return x * jax.nn.sigmoid(x)


_SC_MESH = plsc.VectorSubcoreMesh(core_axis_name="c", subcore_axis_name="s")


def _sc_prep_body(posT_hbm, z_hbm, row_hbm, col_hbm, emb_hbm,
                  x0_hbm, dd_hbm,
                  posT_v, zidx_v, ridx_v, cidx_v, dd_v, rows_v, sem):
    c = lax.axis_index("c")
    s = lax.axis_index("s")
    w = c * NS + s

    pltpu.sync_copy(posT_hbm, posT_v)

    for j in range(5):
        base = w * N_PER_W + j * 64
        pltpu.sync_copy(z_hbm.at[pl.ds(base, 64)], zidx_v)
        pltpu.async_copy(emb_hbm.at[zidx_v], rows_v, sem).wait()
        pltpu.sync_copy(rows_v, x0_hbm.at[pl.ds(base, 64)])

    def dd_chunk(j, carry):
        base = w * E_PER_W + j * CHUNK
        pltpu.sync_copy(row_hbm.at[pl.ds(base, CHUNK)], ridx_v)
        pltpu.sync_copy(col_hbm.at[pl.ds(base, CHUNK)], cidx_v)
        for g in range(CHUNK // 16):
            r = ridx_v[pl.ds(g * 16, 16)]
            cc = cidx_v[pl.ds(g * 16, 16)]
            acc = jnp.zeros((16,), _f32)
            for d in range(3):
                off = jnp.full((16,), d * N_NODES, _i32)
                xr = plsc.load_gather(posT_v, [off + r])
                xc = plsc.load_gather(posT_v, [off + cc])
                df = xr - xc
                acc = acc + df * df
            dd_v[pl.ds(g * 16, 16)] = acc
        pltpu.sync_copy(dd_v, dd_hbm.at[pl.ds(base, CHUNK)])
        return carry

    lax.fori_loop(0, N_CHUNKS, dd_chunk, 0)


_sc_prep = pl.kernel(
    _sc_prep_body,
    out_type=[
        jax.ShapeDtypeStruct((NP, DIM), _f32),
        jax.ShapeDtypeStruct((EP,), _f32),
    ],
    mesh=_SC_MESH,
    scratch_types=[
        pltpu.VMEM((3 * N_NODES,), _f32),
        pltpu.VMEM((64,), _i32),
        pltpu.VMEM((CHUNK,), _i32),
        pltpu.VMEM((CHUNK,), _i32),
        pltpu.VMEM((CHUNK,), _f32),
        pltpu.VMEM((64, DIM), _f32),
        pltpu.SemaphoreType.DMA,
    ],
    compiler_params=pltpu.CompilerParams(needs_layout_passes=False),
)


def _sc_gather_body(p_hbm, q_hbm, row_hbm, col_hbm,
                    s_hbm,
                    cidx0, ridx0, bp0, bq0,
                    cidx1, ridx1, bp1, bq1,
                    semg0, semg1, semw0, semw1):
    c = lax.axis_index("c")
    s = lax.axis_index("s")
    w = c * NS + s
    wbase = w * E_PER_W

    slots = ((cidx0, ridx0, bp0, bq0, semg0, semw0),
             (cidx1, ridx1, bp1, bq1, semg1, semw1))

    def start(j, b):
        cidx, ridx, bp, bq, semg, _ = slots[b]
        base = wbase + j * CHUNK
        pltpu.sync_copy(col_hbm.at[pl.ds(base, CHUNK)], cidx)
        pltpu.sync_copy(row_hbm.at[pl.ds(base, CHUNK)], ridx)
        pltpu.async_copy(p_hbm.at[cidx], bp, semg)
        pltpu.async_copy(q_hbm.at[ridx], bq, semg)

    def finish(j, b):
        cidx, ridx, bp, bq, semg, semw = slots[b]
        base = wbase + j * CHUNK
        pltpu.make_async_copy(p_hbm.at[cidx], bp, semg).wait()
        pltpu.make_async_copy(q_hbm.at[ridx], bq, semg).wait()

        def add_row(r, carry2):
            for g in range(HALF // 16):
                sl = pl.ds(g * 16, 16)
                a = plsc.bitcast(bp[r, sl], _bf16)
                b = plsc.bitcast(bq[r, sl], _bf16)
                bp[r, sl] = plsc.bitcast(a + b, _i32)
            return carry2

        lax.fori_loop(0, CHUNK, add_row, 0)
        pltpu.async_copy(bp, s_hbm.at[pl.ds(base, CHUNK)], semw)

    def wait_wb(j, b):
        _, _, bp, _, _, semw = slots[b]
        base = wbase + j * CHUNK
        pltpu.make_async_copy(bp, s_hbm.at[pl.ds(base, CHUNK)], semw).wait()

    start(0, 0)
    start(1, 1)

    def pair(jj, carry):
        j0 = 2 * jj
        finish(j0, 0)
        wait_wb(j0, 0)
        start(j0 + 2, 0)
        finish(j0 + 1, 1)
        wait_wb(j0 + 1, 1)
        start(j0 + 3, 1)
        return carry

    lax.fori_loop(0, N_CHUNKS // 2 - 1, pair, 0)
    finish(N_CHUNKS - 2, 0)
    finish(N_CHUNKS - 1, 1)
    wait_wb(N_CHUNKS - 2, 0)
    wait_wb(N_CHUNKS - 1, 1)


_sc_gather = pl.kernel(
    _sc_gather_body,
    out_type=jax.ShapeDtypeStruct((EP, HALF), _i32),
    mesh=_SC_MESH,
    scratch_types=[
        pltpu.VMEM((CHUNK,), _i32),
        pltpu.VMEM((CHUNK,), _i32),
        pltpu.VMEM((CHUNK, HALF), _i32),
        pltpu.VMEM((CHUNK, HALF), _i32),
        pltpu.VMEM((CHUNK,), _i32),
        pltpu.VMEM((CHUNK,), _i32),
        pltpu.VMEM((CHUNK, HALF), _i32),
        pltpu.VMEM((CHUNK, HALF), _i32),
        pltpu.SemaphoreType.DMA,
        pltpu.SemaphoreType.DMA,
        pltpu.SemaphoreType.DMA,
        pltpu.SemaphoreType.DMA,
    ],
    compiler_params=pltpu.CompilerParams(needs_layout_passes=False),
)


def _sc_scatter_body(m_hbm, sidx_hbm,
                     agg_hbm,
                     idx0, mb0, idx1, mb1, zbuf, acc_sh, semm0, semm1):
    c = lax.axis_index("c")
    s = lax.axis_index("s")

    def zrow(r, carry):
        for g in range(HALF // 16):
            zbuf[r, pl.ds(g * 16, 16)] = jnp.zeros((16,), _f32)
        return carry

    lax.fori_loop(0, 64, zrow, 0)
    rows_per_sub = ACC_ROWS // NS
    for k in range(rows_per_sub // 64):
        pltpu.sync_copy(zbuf, acc_sh.at[pl.ds(s * rows_per_sub + k * 64, 64)])
    plsc.subcore_barrier()

    e_per_sub = EP // NS
    n_chunks = e_per_sub // CHUNK
    sbase = s * e_per_sub

    slots = ((idx0, mb0, semm0), (idx1, mb1, semm1))

    def startm(j, b):
        idx, mb, semm = slots[b]
        base = sbase + j * CHUNK
        pltpu.sync_copy(sidx_hbm.at[pl.ds(base, CHUNK)], idx)
        pltpu.async_copy(m_hbm.at[c, pl.ds(base, CHUNK)], mb, semm)

    def finishm(j, b):
        idx, mb, semm = slots[b]
        base = sbase + j * CHUNK
        pltpu.make_async_copy(m_hbm.at[c, pl.ds(base, CHUNK)], mb, semm).wait()
        pltpu.sync_copy(mb, acc_sh.at[idx], add=True)

    startm(0, 0)
    startm(1, 1)

    def pair(jj, carry):
        j0 = 2 * jj
        finishm(j0, 0)

        @pl.when(j0 + 2 < n_chunks)
        def _():
            startm(j0 + 2, 0)

        finishm(j0 + 1, 1)

        @pl.when(j0 + 3 < n_chunks)
        def _():
            startm(j0 + 3, 1)

        return carry

    lax.fori_loop(0, n_chunks // 2, pair, 0)
    plsc.subcore_barrier()

    out_per_sub = NP // NS
    pltpu.sync_copy(acc_sh.at[pl.ds(s * out_per_sub, out_per_sub)],
                    agg_hbm.at[c, pl.ds(s * out_per_sub, out_per_sub)])


_sc_scatter = pl.kernel(
    _sc_scatter_body,
    out_type=jax.ShapeDtypeStruct((NC, NP, HALF), _f32),
    mesh=_SC_MESH,
    scratch_types=[
        pltpu.VMEM((CHUNK,), _i32),
        pltpu.VMEM((CHUNK, HALF), _f32),
        pltpu.VMEM((CHUNK,), _i32),
        pltpu.VMEM((CHUNK, HALF), _f32),
        pltpu.VMEM((64, HALF), _f32),
        pltpu.VMEM_SHARED((ACC_ROWS, HALF), _f32),
        pltpu.SemaphoreType.DMA,
        pltpu.SemaphoreType.DMA,
    ],
)


EBLK = 512
NBLK = 256


def _pack_tc(a):
    lo = lax.bitcast_convert_type(
        a[:, :HALF].astype(_bf16).astype(_f32), _i32)
    hi = lax.bitcast_convert_type(
        a[:, HALF:].astype(_bf16).astype(_f32), _i32)
    return jnp.bitwise_or(lax.shift_right_logical(lo, jnp.int32(16)),
                          jnp.bitwise_and(hi, jnp.int32(-65536)))


def _unpack_tc(w):
    lo = lax.bitcast_convert_type(lax.shift_left(w, jnp.int32(16)), _f32)
    hi = lax.bitcast_convert_type(jnp.bitwise_and(w, jnp.int32(-65536)), _f32)
    return jnp.concatenate([lo, hi], axis=1)


def _tc_pq_body(x_ref, wa_ref, wb_ref, p_ref, q_ref):
    x = x_ref[...]
    p_ref[...] = _pack_tc(jnp.dot(x, wa_ref[...], preferred_element_type=_f32))
    q_ref[...] = _pack_tc(jnp.dot(x, wb_ref[...], preferred_element_type=_f32))


def _tc_pq(x, wa, wb):
    return pl.pallas_call(
        _tc_pq_body,
        grid=(NP // NBLK,),
        in_specs=[
            pl.BlockSpec((NBLK, DIM), lambda i: (i, 0)),
            pl.BlockSpec((DIM, DIM), lambda i: (0, 0)),
            pl.BlockSpec((DIM, DIM), lambda i: (0, 0)),
        ],
        out_specs=[
            pl.BlockSpec((NBLK, HALF), lambda i: (i, 0)),
            pl.BlockSpec((NBLK, HALF), lambda i: (i, 0)),
        ],
        out_shape=[
            jax.ShapeDtypeStruct((NP, HALF), _i32),
            jax.ShapeDtypeStruct((NP, HALF), _i32),
        ],
    )(x, wa, wb)


def _bessel_block(dd, freq):
    dist = jnp.sqrt(dd + 1e-12)
    ds = dist / CUTOFF
    p = 6.0
    a = -(p + 1.0) * (p + 2.0) / 2.0
    b = p * (p + 2.0)
    c = -p * (p + 1.0) / 2.0
    ds2 = ds * ds
    ds4 = ds2 * ds2
    ds5 = ds4 * ds
    ds6 = ds5 * ds
    ds7 = ds6 * ds
    env = (1.0 / ds + a * ds5 + b * ds6 + c * ds7) * (ds < 1.0).astype(_f32)
    return env * jnp.sin(freq * ds)


def _tc_edge_body(s_ref, dd_ref, freq_ref, w1c_ref, b1_ref,
                  w2_ref, b2_ref, w3_ref, b3_ref, m_ref):
    ea = _bessel_block(dd_ref[...], freq_ref[...])
    h = _unpack_tc(s_ref[...]) + lax.dot_general(
        ea, w1c_ref[...], (((1,), (0,)), ((), ())),
        preferred_element_type=_f32) + b1_ref[...]
    h = _silu(h)
    h = _silu(jnp.dot(h.astype(_bf16), w2_ref[...],
                      preferred_element_type=_f32) + b2_ref[...])
    m = jnp.dot(h.astype(_bf16), w3_ref[...],
                preferred_element_type=_f32) + b3_ref[...]
    m_ref[0] = m[:, :HALF]
    m_ref[1] = m[:, HALF:]


def _tc_edge(s, dd2, freq2, w1c, b1, w2, b2, w3, b3):
    return pl.pallas_call(
        _tc_edge_body,
        grid=(EP // EBLK,),
        in_specs=[
            pl.BlockSpec((EBLK, HALF), lambda i: (i, 0)),
            pl.BlockSpec((EBLK, 1), lambda i: (i, 0)),
            pl.BlockSpec((1, NUM_BASIS), lambda i: (0, 0)),
            pl.BlockSpec((NUM_BASIS, DIM), lambda i: (0, 0)),
            pl.BlockSpec((1, DIM), lambda i: (0, 0)),
            pl.BlockSpec((DIM, DIM), lambda i: (0, 0)),
            pl.BlockSpec((1, DIM), lambda i: (0, 0)),
            pl.BlockSpec((DIM, DIM), lambda i: (0, 0)),
            pl.BlockSpec((1, DIM), lambda i: (0, 0)),
        ],
        out_specs=pl.BlockSpec((NC, EBLK, HALF), lambda i: (0, i, 0)),
        out_shape=jax.ShapeDtypeStruct((NC, EP, HALF), _f32),
    )(s, dd2, freq2, w1c, b1, w2, b2, w3, b3)


def _tc_upd_body(emit_pq, x_ref, agg_ref, u1x_ref, u1g_ref, ub1_ref,
                 u2_ref, ub2_ref, u3_ref, ub3_ref, *rest):
    if emit_pq:
        wa_ref, wb_ref, xn_ref, p_ref, q_ref = rest
    else:
        (xn_ref,) = rest
    x = x_ref[...]
    aggc = jnp.concatenate([agg_ref[0], agg_ref[1]], axis=1)
    h = (jnp.dot(x, u1x_ref[...], preferred_element_type=_f32)
         + jnp.dot(aggc, u1g_ref[...], preferred_element_type=_f32)
         + ub1_ref[...])
    h = _silu(h)
    h = _silu(jnp.dot(h, u2_ref[...], preferred_element_type=_f32) + ub2_ref[...])
    xn = x + jnp.dot(h, u3_ref[...], preferred_element_type=_f32) + ub3_ref[...]
    xn_ref[...] = xn
    if emit_pq:
        p_ref[...] = _pack_tc(jnp.dot(xn, wa_ref[...],
                                      preferred_element_type=_f32))
        q_ref[...] = _pack_tc(jnp.dot(xn, wb_ref[...],
                                      preferred_element_type=_f32))


def _tc_upd(x, agg, u1x, u1g, ub1, u2, ub2, u3, ub3, wa=None, wb=None):
    emit_pq = wa is not None
    full = lambda i: (0, 0)
    in_specs = [
        pl.BlockSpec((NBLK, DIM), lambda i: (i, 0)),
        pl.BlockSpec((NC, NBLK, HALF), lambda i: (0, i, 0)),
        pl.BlockSpec((DIM, DIM), full),
        pl.BlockSpec((DIM, DIM), full),
        pl.BlockSpec((1, DIM), full),
        pl.BlockSpec((DIM, DIM), full),
        pl.BlockSpec((1, DIM), full),
        pl.BlockSpec((DIM, DIM), full),
        pl.BlockSpec((1, DIM), full),
    ]
    args = [x, agg, u1x, u1g, ub1, u2, ub2, u3, ub3]
    nblk = pl.BlockSpec((NBLK, DIM), lambda i: (i, 0))
    hblk = pl.BlockSpec((NBLK, HALF), lambda i: (i, 0))
    nshape = jax.ShapeDtypeStruct((NP, DIM), _f32)
    hshape = jax.ShapeDtypeStruct((NP, HALF), _i32)
    if emit_pq:
        in_specs += [pl.BlockSpec((DIM, DIM), full), pl.BlockSpec((DIM, DIM), full)]
        args += [wa, wb]
        out_specs = [nblk, hblk, hblk]
        out_shape = [nshape, hshape, hshape]
    else:
        out_specs = [nblk]
        out_shape = [nshape]
    return pl.pallas_call(
        functools.partial(_tc_upd_body, emit_pq),
        grid=(NP // NBLK,),
        in_specs=in_specs,
        out_specs=out_specs,
        out_shape=out_shape,
    )(*args)


def _tc_tail_body(x_ref, batch_ref, h1_ref, hb1_ref, h2_ref, hb2_ref,
                  h3_ref, hb3_ref, out_ref, acc_s, cnt_s):
    i = pl.program_id(0)

    @pl.when(i == 0)
    def _init():
        acc_s[...] = jnp.zeros_like(acc_s)
        cnt_s[...] = jnp.zeros_like(cnt_s)

    gids = lax.broadcasted_iota(_i32, (1, N_GRAPHS), 1)
    oh = (batch_ref[...] == gids).astype(_f32)
    x = x_ref[...]
    acc_s[...] += lax.dot_general(oh, x, (((0,), (0,)), ((), ())),
                                  preferred_element_type=_f32)
    ones = jnp.ones((NBLK, 1), _f32)
    cnt_s[...] += lax.dot_general(oh, ones, (((0,), (0,)), ((), ())),
                                  preferred_element_type=_f32)

    @pl.when(i == pl.num_programs(0) - 1)
    def _final():
        pooled = acc_s[...] / jnp.maximum(cnt_s[...], 1.0)
        h = _silu(jnp.dot(pooled, h1_ref[...], preferred_element_type=_f32)
                  + hb1_ref[...])
        h = _silu(jnp.dot(h, h2_ref[...], preferred_element_type=_f32)
                  + hb2_ref[...])
        out_ref[...] = (jnp.dot(h, h3_ref[...], preferred_element_type=_f32)
                        + hb3_ref[...])


def _tc_tail(x, batch2, h1, hb1, h2, hb2, h3, hb3):
    full = lambda i: (0, 0)
    return pl.pallas_call(
        _tc_tail_body,
        grid=(NP // NBLK,),
        in_specs=[
            pl.BlockSpec((NBLK, DIM), lambda i: (i, 0)),
            pl.BlockSpec((NBLK, 1), lambda i: (i, 0)),
            pl.BlockSpec((DIM, DIM), full),
            pl.BlockSpec((1, DIM), full),
            pl.BlockSpec((DIM, DIM), full),
            pl.BlockSpec((1, DIM), full),
            pl.BlockSpec((DIM, 1), full),
            pl.BlockSpec((1, 1), full),
        ],
        out_specs=pl.BlockSpec((N_GRAPHS, 1), full),
        out_shape=jax.ShapeDtypeStruct((N_GRAPHS, 1), _f32),
        scratch_shapes=[
            pltpu.VMEM((N_GRAPHS, DIM), _f32),
            pltpu.VMEM((N_GRAPHS, 1), _f32),
        ],
    )(x, batch2, h1, hb1, h2, hb2, h3, hb3)


def kernel(z, edge_index, batch, pos, emb, freq, layers, head):
    row = edge_index[0].astype(_i32)
    col = edge_index[1].astype(_i32)
    rowp = jnp.pad(row, (0, EP - N_EDGES))
    colp = jnp.pad(col, (0, EP - N_EDGES))
    sidx = jnp.pad(col, (0, EP - N_EDGES), constant_values=N_NODES)
    zp = jnp.pad(z.astype(_i32), (0, NP - N_NODES))
    batchp = jnp.pad(batch.astype(_i32), (0, NP - N_NODES),
                     constant_values=N_GRAPHS).reshape(NP, 1)
    posT = pos.T.reshape(3 * N_NODES)
    freq2 = freq.reshape(1, NUM_BASIS)

    x0, dd = _sc_prep(posT, zp, rowp, colp, emb)
    dd2 = dd.reshape(EP, 1)

    def msg_parts(layer):
        (w1, b1), (w2, b2), (w3, b3) = layer["msg"]
        return (w1[:DIM], w1[DIM:2 * DIM], w1[2 * DIM:],
                b1.reshape(1, DIM), w2.astype(_bf16), b2.reshape(1, DIM),
                w3.astype(_bf16), b3.reshape(1, DIM))

    def upd_parts(layer):
        (u1, ub1), (u2, ub2), (u3, ub3) = layer["upd"]
        return (u1[:DIM], u1[DIM:], ub1.reshape(1, DIM), u2,
                ub2.reshape(1, DIM), u3, ub3.reshape(1, DIM))

    x = x0
    wa0, wb0 = msg_parts(layers[0])[0], msg_parts(layers[0])[1]
    p, q = _tc_pq(x0, wa0, wb0)
    for li, layer in enumerate(layers):
        _, _, w1c, b1, w2, b2, w3, b3 = msg_parts(layer)
        s = _sc_gather(p, q, rowp, colp)
        m = _tc_edge(s, dd2, freq2, w1c, b1, w2, b2, w3, b3)
        agg = _sc_scatter(m, sidx)
        u1x, u1g, ub1, u2, ub2, u3, ub3 = upd_parts(layer)
        if li + 1 < len(layers):
            wa, wb = msg_parts(layers[li + 1])[0], msg_parts(layers[li + 1])[1]
            x, p, q = _tc_upd(x, agg, u1x, u1g, ub1, u2, ub2, u3, ub3, wa, wb)
        else:
            (x,) = _tc_upd(x, agg, u1x, u1g, ub1, u2, ub2, u3, ub3)

    (h1, hb1), (h2, hb2), (h3, hb3) = head
    return _tc_tail(x, batchp, h1, hb1.reshape(1, DIM), h2,
                    hb2.reshape(1, DIM), h3, hb3.reshape(1, 1))


# speedup vs baseline: 1.0528x; 1.0528x over previous
import functools

import jax
import jax.numpy as jnp
from jax import lax
from jax.experimental import pallas as pl
from jax.experimental.pallas import tpu as pltpu
from jax.experimental.pallas import tpu_sc as plsc

N_NODES = 10000
N_EDGES = 160000
DIM = 256
HALF = 128
NUM_BASIS = 16
CUTOFF = 5.0
N_GRAPHS = 8

NC = 2
NS = 16
NW = NC * NS
NP = 10240
EP = 163840
CHUNK = 128
E_PER_W = EP // NW
N_CHUNKS = E_PER_W // CHUNK
N_PER_W = NP // NW
ACC_ROWS = NP

_f32 = jnp.float32
_i32 = jnp.int32
_bf16 = jnp.bfloat16


def _silu(x):
    return x * jax.nn.sigmoid(x)


_SC_MESH = plsc.VectorSubcoreMesh(core_axis_name="c", subcore_axis_name="s")


def _sc_prep_body(posT_hbm, z_hbm, row_hbm, col_hbm, emb_hbm,
                  x0_hbm, dd_hbm,
                  posT_v, zidx_v, ridx_v, cidx_v, dd_v, rows_v, sem):
    c = lax.axis_index("c")
    s = lax.axis_index("s")
    w = c * NS + s

    pltpu.sync_copy(posT_hbm, posT_v)

    for j in range(5):
        base = w * N_PER_W + j * 64
        pltpu.sync_copy(z_hbm.at[pl.ds(base, 64)], zidx_v)
        pltpu.async_copy(emb_hbm.at[zidx_v], rows_v, sem).wait()
        pltpu.sync_copy(rows_v, x0_hbm.at[pl.ds(base, 64)])

    def dd_chunk(j, carry):
        base = w * E_PER_W + j * CHUNK
        pltpu.sync_copy(row_hbm.at[pl.ds(base, CHUNK)], ridx_v)
        pltpu.sync_copy(col_hbm.at[pl.ds(base, CHUNK)], cidx_v)
        for g in range(CHUNK // 16):
            r = ridx_v[pl.ds(g * 16, 16)]
            cc = cidx_v[pl.ds(g * 16, 16)]
            acc = jnp.zeros((16,), _f32)
            for d in range(3):
                off = jnp.full((16,), d * N_NODES, _i32)
                xr = plsc.load_gather(posT_v, [off + r])
                xc = plsc.load_gather(posT_v, [off + cc])
                df = xr - xc
                acc = acc + df * df
            dd_v[pl.ds(g * 16, 16)] = acc
        pltpu.sync_copy(dd_v, dd_hbm.at[pl.ds(base, CHUNK)])
        return carry

    lax.fori_loop(0, N_CHUNKS, dd_chunk, 0)


_sc_prep = pl.kernel(
    _sc_prep_body,
    out_type=[
        jax.ShapeDtypeStruct((NP, DIM), _f32),
        jax.ShapeDtypeStruct((EP,), _f32),
    ],
    mesh=_SC_MESH,
    scratch_types=[
        pltpu.VMEM((3 * N_NODES,), _f32),
        pltpu.VMEM((64,), _i32),
        pltpu.VMEM((CHUNK,), _i32),
        pltpu.VMEM((CHUNK,), _i32),
        pltpu.VMEM((CHUNK,), _f32),
        pltpu.VMEM((64, DIM), _f32),
        pltpu.SemaphoreType.DMA,
    ],
    compiler_params=pltpu.CompilerParams(needs_layout_passes=False),
)


def _sc_gather_body(p_hbm, q_hbm, row_hbm, col_hbm,
                    s_hbm,
                    cidx0, ridx0, bp0, bq0,
                    cidx1, ridx1, bp1, bq1,
                    semg0, semg1, semw0, semw1):
    c = lax.axis_index("c")
    s = lax.axis_index("s")
    w = c * NS + s
    wbase = w * E_PER_W

    slots = ((cidx0, ridx0, bp0, bq0, semg0, semw0),
             (cidx1, ridx1, bp1, bq1, semg1, semw1))

    def start(j, b):
        cidx, ridx, bp, bq, semg, _ = slots[b]
        base = wbase + j * CHUNK
        pltpu.sync_copy(col_hbm.at[pl.ds(base, CHUNK)], cidx)
        pltpu.sync_copy(row_hbm.at[pl.ds(base, CHUNK)], ridx)
        pltpu.async_copy(p_hbm.at[cidx], bp, semg)
        pltpu.async_copy(q_hbm.at[ridx], bq, semg)

    def finish(j, b):
        cidx, ridx, bp, bq, semg, semw = slots[b]
        base = wbase + j * CHUNK
        pltpu.make_async_copy(p_hbm.at[cidx], bp, semg).wait()
        pltpu.make_async_copy(q_hbm.at[ridx], bq, semg).wait()

        def add_row(r, carry2):
            for g in range(HALF // 16):
                sl = pl.ds(g * 16, 16)
                a = plsc.bitcast(bp[r, sl], _bf16)
                b = plsc.bitcast(bq[r, sl], _bf16)
                bp[r, sl] = plsc.bitcast(a + b, _i32)
            return carry2

        lax.fori_loop(0, CHUNK, add_row, 0)
        pltpu.async_copy(bp, s_hbm.at[pl.ds(base, CHUNK)], semw)

    def wait_wb(j, b):
        _, _, bp, _, _, semw = slots[b]
        base = wbase + j * CHUNK
        pltpu.make_async_copy(bp, s_hbm.at[pl.ds(base, CHUNK)], semw).wait()

    start(0, 0)
    start(1, 1)

    def pair(jj, carry):
        j0 = 2 * jj
        finish(j0, 0)
        wait_wb(j0, 0)
        start(j0 + 2, 0)
        finish(j0 + 1, 1)
        wait_wb(j0 + 1, 1)
        start(j0 + 3, 1)
        return carry

    lax.fori_loop(0, N_CHUNKS // 2 - 1, pair, 0)
    finish(N_CHUNKS - 2, 0)
    finish(N_CHUNKS - 1, 1)
    wait_wb(N_CHUNKS - 2, 0)
    wait_wb(N_CHUNKS - 1, 1)


_sc_gather = pl.kernel(
    _sc_gather_body,
    out_type=jax.ShapeDtypeStruct((EP, HALF), _i32),
    mesh=_SC_MESH,
    scratch_types=[
        pltpu.VMEM((CHUNK,), _i32),
        pltpu.VMEM((CHUNK,), _i32),
        pltpu.VMEM((CHUNK, HALF), _i32),
        pltpu.VMEM((CHUNK, HALF), _i32),
        pltpu.VMEM((CHUNK,), _i32),
        pltpu.VMEM((CHUNK,), _i32),
        pltpu.VMEM((CHUNK, HALF), _i32),
        pltpu.VMEM((CHUNK, HALF), _i32),
        pltpu.SemaphoreType.DMA,
        pltpu.SemaphoreType.DMA,
        pltpu.SemaphoreType.DMA,
        pltpu.SemaphoreType.DMA,
    ],
    compiler_params=pltpu.CompilerParams(needs_layout_passes=False),
)


EHALF = EP // 2


def _sc_scatter_body(off, m_hbm, sidx_hbm,
                     agg_hbm,
                     idx0, mb0, idx1, mb1, zbuf, acc_sh, semm0, semm1):
    c = lax.axis_index("c")
    s = lax.axis_index("s")

    def zrow(r, carry):
        for g in range(HALF // 16):
            zbuf[r, pl.ds(g * 16, 16)] = jnp.zeros((16,), _f32)
        return carry

    lax.fori_loop(0, 64, zrow, 0)
    rows_per_sub = ACC_ROWS // NS
    for k in range(rows_per_sub // 64):
        pltpu.sync_copy(zbuf, acc_sh.at[pl.ds(s * rows_per_sub + k * 64, 64)])
    plsc.subcore_barrier()

    e_per_sub = EHALF // NS
    n_chunks = e_per_sub // CHUNK
    sbase = s * e_per_sub

    slots = ((idx0, mb0, semm0), (idx1, mb1, semm1))

    def startm(j, b):
        idx, mb, semm = slots[b]
        base = sbase + j * CHUNK
        pltpu.sync_copy(sidx_hbm.at[pl.ds(off + base, CHUNK)], idx)
        pltpu.async_copy(m_hbm.at[c, pl.ds(base, CHUNK)], mb, semm)

    def finishm(j, b):
        idx, mb, semm = slots[b]
        base = sbase + j * CHUNK
        pltpu.make_async_copy(m_hbm.at[c, pl.ds(base, CHUNK)], mb, semm).wait()
        pltpu.sync_copy(mb, acc_sh.at[idx], add=True)

    startm(0, 0)
    startm(1, 1)

    def pair(jj, carry):
        j0 = 2 * jj
        finishm(j0, 0)
        startm(j0 + 2, 0)
        finishm(j0 + 1, 1)
        startm(j0 + 3, 1)
        return carry

    lax.fori_loop(0, n_chunks // 2 - 1, pair, 0)
    finishm(n_chunks - 2, 0)
    finishm(n_chunks - 1, 1)
    plsc.subcore_barrier()

    out_per_sub = NP // NS
    pltpu.sync_copy(acc_sh.at[pl.ds(s * out_per_sub, out_per_sub)],
                    agg_hbm.at[c, pl.ds(s * out_per_sub, out_per_sub)])


def _make_scatter(off):
    return pl.kernel(
        functools.partial(_sc_scatter_body, off),
        out_type=jax.ShapeDtypeStruct((NC, NP, HALF), _f32),
        mesh=_SC_MESH,
        scratch_types=[
            pltpu.VMEM((CHUNK,), _i32),
            pltpu.VMEM((CHUNK, HALF), _f32),
            pltpu.VMEM((CHUNK,), _i32),
            pltpu.VMEM((CHUNK, HALF), _f32),
            pltpu.VMEM((64, HALF), _f32),
            pltpu.VMEM_SHARED((ACC_ROWS, HALF), _f32),
            pltpu.SemaphoreType.DMA,
            pltpu.SemaphoreType.DMA,
        ],
    )


_sc_scatter0 = _make_scatter(0)
_sc_scatter1 = _make_scatter(EHALF)


EBLK = 512
NBLK = 256


def _pack_tc(a):
    lo = lax.bitcast_convert_type(
        a[:, :HALF].astype(_bf16).astype(_f32), _i32)
    hi = lax.bitcast_convert_type(
        a[:, HALF:].astype(_bf16).astype(_f32), _i32)
    return jnp.bitwise_or(lax.shift_right_logical(lo, jnp.int32(16)),
                          jnp.bitwise_and(hi, jnp.int32(-65536)))


def _unpack_tc(w):
    lo = lax.bitcast_convert_type(lax.shift_left(w, jnp.int32(16)), _f32)
    hi = lax.bitcast_convert_type(jnp.bitwise_and(w, jnp.int32(-65536)), _f32)
    return jnp.concatenate([lo, hi], axis=1)


def _tc_pq_body(x_ref, wa_ref, wb_ref, p_ref, q_ref):
    x = x_ref[...]
    p_ref[...] = _pack_tc(jnp.dot(x, wa_ref[...], preferred_element_type=_f32))
    q_ref[...] = _pack_tc(jnp.dot(x, wb_ref[...], preferred_element_type=_f32))


def _tc_pq(x, wa, wb):
    return pl.pallas_call(
        _tc_pq_body,
        grid=(NP // NBLK,),
        in_specs=[
            pl.BlockSpec((NBLK, DIM), lambda i: (i, 0)),
            pl.BlockSpec((DIM, DIM), lambda i: (0, 0)),
            pl.BlockSpec((DIM, DIM), lambda i: (0, 0)),
        ],
        out_specs=[
            pl.BlockSpec((NBLK, HALF), lambda i: (i, 0)),
            pl.BlockSpec((NBLK, HALF), lambda i: (i, 0)),
        ],
        out_shape=[
            jax.ShapeDtypeStruct((NP, HALF), _i32),
            jax.ShapeDtypeStruct((NP, HALF), _i32),
        ],
    )(x, wa, wb)


def _bessel_block(dd, freq):
    dist = jnp.sqrt(dd + 1e-12)
    ds = dist / CUTOFF
    p = 6.0
    a = -(p + 1.0) * (p + 2.0) / 2.0
    b = p * (p + 2.0)
    c = -p * (p + 1.0) / 2.0
    ds2 = ds * ds
    ds4 = ds2 * ds2
    ds5 = ds4 * ds
    ds6 = ds5 * ds
    ds7 = ds6 * ds
    env = (1.0 / ds + a * ds5 + b * ds6 + c * ds7) * (ds < 1.0).astype(_f32)
    return env * jnp.sin(freq * ds)


def _tc_edge_body(s_ref, dd_ref, freq_ref, w1c_ref, b1_ref,
                  w2_ref, b2_ref, w3_ref, b3_ref, m_ref):
    ea = _bessel_block(dd_ref[...], freq_ref[...])
    h = _unpack_tc(s_ref[...]) + lax.dot_general(
        ea, w1c_ref[...], (((1,), (0,)), ((), ())),
        preferred_element_type=_f32) + b1_ref[...]
    h = _silu(h)
    h = _silu(jnp.dot(h, w2_ref[...], preferred_element_type=_f32) + b2_ref[...])
    m = jnp.dot(h, w3_ref[...], preferred_element_type=_f32) + b3_ref[...]
    m_ref[0] = m[:, :HALF]
    m_ref[1] = m[:, HALF:]


def _tc_edge(s, dd2, freq2, w1c, b1, w2, b2, w3, b3, half):
    nblk = EHALF // EBLK
    off = half * nblk
    return pl.pallas_call(
        _tc_edge_body,
        grid=(nblk,),
        in_specs=[
            pl.BlockSpec((EBLK, HALF), lambda i: (i + off, 0)),
            pl.BlockSpec((EBLK, 1), lambda i: (i + off, 0)),
            pl.BlockSpec((1, NUM_BASIS), lambda i: (0, 0)),
            pl.BlockSpec((NUM_BASIS, DIM), lambda i: (0, 0)),
            pl.BlockSpec((1, DIM), lambda i: (0, 0)),
            pl.BlockSpec((DIM, DIM), lambda i: (0, 0)),
            pl.BlockSpec((1, DIM), lambda i: (0, 0)),
            pl.BlockSpec((DIM, DIM), lambda i: (0, 0)),
            pl.BlockSpec((1, DIM), lambda i: (0, 0)),
        ],
        out_specs=pl.BlockSpec((NC, EBLK, HALF), lambda i: (0, i, 0)),
        out_shape=jax.ShapeDtypeStruct((NC, EHALF, HALF), _f32),
    )(s, dd2, freq2, w1c, b1, w2, b2, w3, b3)


def _tc_upd_body(emit_pq, x_ref, agg1_ref, agg2_ref, u1x_ref, u1g_ref,
                 ub1_ref, u2_ref, ub2_ref, u3_ref, ub3_ref, *rest):
    if emit_pq:
        wa_ref, wb_ref, xn_ref, p_ref, q_ref = rest
    else:
        (xn_ref,) = rest
    x = x_ref[...]
    aggc = jnp.concatenate([agg1_ref[0] + agg2_ref[0],
                            agg1_ref[1] + agg2_ref[1]], axis=1)
    h = (jnp.dot(x, u1x_ref[...], preferred_element_type=_f32)
         + jnp.dot(aggc, u1g_ref[...], preferred_element_type=_f32)
         + ub1_ref[...])
    h = _silu(h)
    h = _silu(jnp.dot(h, u2_ref[...], preferred_element_type=_f32) + ub2_ref[...])
    xn = x + jnp.dot(h, u3_ref[...], preferred_element_type=_f32) + ub3_ref[...]
    xn_ref[...] = xn
    if emit_pq:
        p_ref[...] = _pack_tc(jnp.dot(xn, wa_ref[...],
                                      preferred_element_type=_f32))
        q_ref[...] = _pack_tc(jnp.dot(xn, wb_ref[...],
                                      preferred_element_type=_f32))


def _tc_upd(x, agg1, agg2, u1x, u1g, ub1, u2, ub2, u3, ub3, wa=None, wb=None):
    emit_pq = wa is not None
    full = lambda i: (0, 0)
    in_specs = [
        pl.BlockSpec((NBLK, DIM), lambda i: (i, 0)),
        pl.BlockSpec((NC, NBLK, HALF), lambda i: (0, i, 0)),
        pl.BlockSpec((NC, NBLK, HALF), lambda i: (0, i, 0)),
        pl.BlockSpec((DIM, DIM), full),
        pl.BlockSpec((DIM, DIM), full),
        pl.BlockSpec((1, DIM), full),
        pl.BlockSpec((DIM, DIM), full),
        pl.BlockSpec((1, DIM), full),
        pl.BlockSpec((DIM, DIM), full),
        pl.BlockSpec((1, DIM), full),
    ]
    args = [x, agg1, agg2, u1x, u1g, ub1, u2, ub2, u3, ub3]
    nblk = pl.BlockSpec((NBLK, DIM), lambda i: (i, 0))
    hblk = pl.BlockSpec((NBLK, HALF), lambda i: (i, 0))
    nshape = jax.ShapeDtypeStruct((NP, DIM), _f32)
    hshape = jax.ShapeDtypeStruct((NP, HALF), _i32)
    if emit_pq:
        in_specs += [pl.BlockSpec((DIM, DIM), full), pl.BlockSpec((DIM, DIM), full)]
        args += [wa, wb]
        out_specs = [nblk, hblk, hblk]
        out_shape = [nshape, hshape, hshape]
    else:
        out_specs = [nblk]
        out_shape = [nshape]
    return pl.pallas_call(
        functools.partial(_tc_upd_body, emit_pq),
        grid=(NP // NBLK,),
        in_specs=in_specs,
        out_specs=out_specs,
        out_shape=out_shape,
    )(*args)


def _tc_tail_body(x_ref, batch_ref, h1_ref, hb1_ref, h2_ref, hb2_ref,
                  h3_ref, hb3_ref, out_ref, acc_s, cnt_s):
    i = pl.program_id(0)

    @pl.when(i == 0)
    def _init():
        acc_s[...] = jnp.zeros_like(acc_s)
        cnt_s[...] = jnp.zeros_like(cnt_s)

    gids = lax.broadcasted_iota(_i32, (1, N_GRAPHS), 1)
    oh = (batch_ref[...] == gids).astype(_f32)
    x = x_ref[...]
    acc_s[...] += lax.dot_general(oh, x, (((0,), (0,)), ((), ())),
                                  preferred_element_type=_f32)
    ones = jnp.ones((NBLK, 1), _f32)
    cnt_s[...] += lax.dot_general(oh, ones, (((0,), (0,)), ((), ())),
                                  preferred_element_type=_f32)

    @pl.when(i == pl.num_programs(0) - 1)
    def _final():
        pooled = acc_s[...] / jnp.maximum(cnt_s[...], 1.0)
        h = _silu(jnp.dot(pooled, h1_ref[...], preferred_element_type=_f32)
                  + hb1_ref[...])
        h = _silu(jnp.dot(h, h2_ref[...], preferred_element_type=_f32)
                  + hb2_ref[...])
        out_ref[...] = (jnp.dot(h, h3_ref[...], preferred_element_type=_f32)
                        + hb3_ref[...])


def _tc_tail(x, batch2, h1, hb1, h2, hb2, h3, hb3):
    full = lambda i: (0, 0)
    return pl.pallas_call(
        _tc_tail_body,
        grid=(NP // NBLK,),
        in_specs=[
            pl.BlockSpec((NBLK, DIM), lambda i: (i, 0)),
            pl.BlockSpec((NBLK, 1), lambda i: (i, 0)),
            pl.BlockSpec((DIM, DIM), full),
            pl.BlockSpec((1, DIM), full),
            pl.BlockSpec((DIM, DIM), full),
            pl.BlockSpec((1, DIM), full),
            pl.BlockSpec((DIM, 1), full),
            pl.BlockSpec((1, 1), full),
        ],
        out_specs=pl.BlockSpec((N_GRAPHS, 1), full),
        out_shape=jax.ShapeDtypeStruct((N_GRAPHS, 1), _f32),
        scratch_shapes=[
            pltpu.VMEM((N_GRAPHS, DIM), _f32),
            pltpu.VMEM((N_GRAPHS, 1), _f32),
        ],
    )(x, batch2, h1, hb1, h2, hb2, h3, hb3)


def kernel(z, edge_index, batch, pos, emb, freq, layers, head):
    row = edge_index[0].astype(_i32)
    col = edge_index[1].astype(_i32)
    rowp = jnp.pad(row, (0, EP - N_EDGES))
    colp = jnp.pad(col, (0, EP - N_EDGES))
    sidx = jnp.pad(col, (0, EP - N_EDGES), constant_values=N_NODES)
    zp = jnp.pad(z.astype(_i32), (0, NP - N_NODES))
    batchp = jnp.pad(batch.astype(_i32), (0, NP - N_NODES),
                     constant_values=N_GRAPHS).reshape(NP, 1)
    posT = pos.T.reshape(3 * N_NODES)
    freq2 = freq.reshape(1, NUM_BASIS)

    x0, dd = _sc_prep(posT, zp, rowp, colp, emb)
    dd2 = dd.reshape(EP, 1)

    def msg_parts(layer):
        (w1, b1), (w2, b2), (w3, b3) = layer["msg"]
        return (w1[:DIM], w1[DIM:2 * DIM], w1[2 * DIM:],
                b1.reshape(1, DIM), w2, b2.reshape(1, DIM), w3,
                b3.reshape(1, DIM))

    def upd_parts(layer):
        (u1, ub1), (u2, ub2), (u3, ub3) = layer["upd"]
        return (u1[:DIM], u1[DIM:], ub1.reshape(1, DIM), u2,
                ub2.reshape(1, DIM), u3, ub3.reshape(1, DIM))

    x = x0
    wa0, wb0 = msg_parts(layers[0])[0], msg_parts(layers[0])[1]
    p, q = _tc_pq(x0, wa0, wb0)
    for li, layer in enumerate(layers):
        _, _, w1c, b1, w2, b2, w3, b3 = msg_parts(layer)
        s = _sc_gather(p, q, rowp, colp)
        m1 = _tc_edge(s, dd2, freq2, w1c, b1, w2, b2, w3, b3, 0)
        agg1 = _sc_scatter0(m1, sidx)
        m2 = _tc_edge(s, dd2, freq2, w1c, b1, w2, b2, w3, b3, 1)
        agg2 = _sc_scatter1(m2, sidx)
        u1x, u1g, ub1, u2, ub2, u3, ub3 = upd_parts(layer)
        if li + 1 < len(layers):
            wa, wb = msg_parts(layers[li + 1])[0], msg_parts(layers[li + 1])[1]
            x, p, q = _tc_upd(x, agg1, agg2, u1x, u1g, ub1, u2, ub2, u3, ub3,
                              wa, wb)
        else:
            (x,) = _tc_upd(x, agg1, agg2, u1x, u1g, ub1, u2, ub2, u3, ub3)

    (h1, hb1), (h2, hb2), (h3, hb3) = head
    return _tc_tail(x, batchp, h1, hb1.reshape(1, DIM), h2,
                    hb2.reshape(1, DIM), h3, hb3.reshape(1, 1))


# speedup vs baseline: 1.1353x; 1.0784x over previous
import functools

import jax
import jax.numpy as jnp
from jax import lax
from jax.experimental import pallas as pl
from jax.experimental.pallas import tpu as pltpu
from jax.experimental.pallas import tpu_sc as plsc

N_NODES = 10000
N_EDGES = 160000
DIM = 256
HALF = 128
NUM_BASIS = 16
CUTOFF = 5.0
N_GRAPHS = 8

NC = 2
NS = 16
NW = NC * NS
NP = 10240
EP = 163840
CHUNK = 128
E_PER_W = EP // NW
N_CHUNKS = E_PER_W // CHUNK
N_PER_W = NP // NW
ACC_ROWS = NP
EHALF = EP // 2

_f32 = jnp.float32
_i32 = jnp.int32
_bf16 = jnp.bfloat16


def _silu(x):
    return x * jax.nn.sigmoid(x)


_SC_MESH = plsc.VectorSubcoreMesh(core_axis_name="c", subcore_axis_name="s")


def _sc_prep_body(posT_hbm, z_hbm, row_hbm, col_hbm, emb_hbm,
                  x0_hbm, dd_hbm,
                  posT_v, zidx_v, ridx_v, cidx_v, dd_v, rows_v, sem):
    c = lax.axis_index("c")
    s = lax.axis_index("s")
    w = c * NS + s

    pltpu.sync_copy(posT_hbm, posT_v)

    for j in range(5):
        base = w * N_PER_W + j * 64
        pltpu.sync_copy(z_hbm.at[pl.ds(base, 64)], zidx_v)
        pltpu.async_copy(emb_hbm.at[zidx_v], rows_v, sem).wait()
        pltpu.sync_copy(rows_v, x0_hbm.at[pl.ds(base, 64)])

    def dd_chunk(j, carry):
        base = w * E_PER_W + j * CHUNK
        pltpu.sync_copy(row_hbm.at[pl.ds(base, CHUNK)], ridx_v)
        pltpu.sync_copy(col_hbm.at[pl.ds(base, CHUNK)], cidx_v)
        for g in range(CHUNK // 16):
            r = ridx_v[pl.ds(g * 16, 16)]
            cc = cidx_v[pl.ds(g * 16, 16)]
            acc = jnp.zeros((16,), _f32)
            for d in range(3):
                off = jnp.full((16,), d * N_NODES, _i32)
                xr = plsc.load_gather(posT_v, [off + r])
                xc = plsc.load_gather(posT_v, [off + cc])
                df = xr - xc
                acc = acc + df * df
            dd_v[pl.ds(g * 16, 16)] = acc
        pltpu.sync_copy(dd_v, dd_hbm.at[pl.ds(base, CHUNK)])
        return carry

    lax.fori_loop(0, N_CHUNKS, dd_chunk, 0)


_sc_prep = pl.kernel(
    _sc_prep_body,
    out_type=[
        jax.ShapeDtypeStruct((NP, DIM), _f32),
        jax.ShapeDtypeStruct((EP,), _f32),
    ],
    mesh=_SC_MESH,
    scratch_types=[
        pltpu.VMEM((3 * N_NODES,), _f32),
        pltpu.VMEM((64,), _i32),
        pltpu.VMEM((CHUNK,), _i32),
        pltpu.VMEM((CHUNK,), _i32),
        pltpu.VMEM((CHUNK,), _f32),
        pltpu.VMEM((64, DIM), _f32),
        pltpu.SemaphoreType.DMA,
    ],
    compiler_params=pltpu.CompilerParams(needs_layout_passes=False),
)


EH_PER_W = EHALF // NW
NH_CHUNKS = EH_PER_W // CHUNK


def _sc_gather_body(off, p_hbm, q_hbm, row_hbm, col_hbm,
                    s_hbm,
                    cidx0, ridx0, bp0, bq0,
                    cidx1, ridx1, bp1, bq1,
                    semg0, semg1, semw0, semw1):
    c = lax.axis_index("c")
    s = lax.axis_index("s")
    w = c * NS + s
    wbase = w * EH_PER_W

    slots = ((cidx0, ridx0, bp0, bq0, semg0, semw0),
             (cidx1, ridx1, bp1, bq1, semg1, semw1))

    def start(j, b):
        cidx, ridx, bp, bq, semg, _ = slots[b]
        base = wbase + j * CHUNK
        pltpu.sync_copy(col_hbm.at[pl.ds(off + base, CHUNK)], cidx)
        pltpu.sync_copy(row_hbm.at[pl.ds(off + base, CHUNK)], ridx)
        pltpu.async_copy(p_hbm.at[cidx], bp, semg)
        pltpu.async_copy(q_hbm.at[ridx], bq, semg)

    def finish(j, b):
        cidx, ridx, bp, bq, semg, semw = slots[b]
        base = wbase + j * CHUNK
        pltpu.make_async_copy(p_hbm.at[cidx], bp, semg).wait()
        pltpu.make_async_copy(q_hbm.at[ridx], bq, semg).wait()

        def add_row(r, carry2):
            for g in range(HALF // 16):
                sl = pl.ds(g * 16, 16)
                a = plsc.bitcast(bp[r, sl], _bf16)
                b = plsc.bitcast(bq[r, sl], _bf16)
                bp[r, sl] = plsc.bitcast(a + b, _i32)
            return carry2

        lax.fori_loop(0, CHUNK, add_row, 0)
        pltpu.async_copy(bp, s_hbm.at[pl.ds(base, CHUNK)], semw)

    def wait_wb(j, b):
        _, _, bp, _, _, semw = slots[b]
        base = wbase + j * CHUNK
        pltpu.make_async_copy(bp, s_hbm.at[pl.ds(base, CHUNK)], semw).wait()

    start(0, 0)
    start(1, 1)

    def pair(jj, carry):
        j0 = 2 * jj
        finish(j0, 0)
        wait_wb(j0, 0)
        start(j0 + 2, 0)
        finish(j0 + 1, 1)
        wait_wb(j0 + 1, 1)
        start(j0 + 3, 1)
        return carry

    lax.fori_loop(0, NH_CHUNKS // 2 - 1, pair, 0)
    finish(NH_CHUNKS - 2, 0)
    finish(NH_CHUNKS - 1, 1)
    wait_wb(NH_CHUNKS - 2, 0)
    wait_wb(NH_CHUNKS - 1, 1)


def _make_gather(off):
    return pl.kernel(
        functools.partial(_sc_gather_body, off),
        out_type=jax.ShapeDtypeStruct((EHALF, HALF), _i32),
        mesh=_SC_MESH,
        scratch_types=[
            pltpu.VMEM((CHUNK,), _i32),
            pltpu.VMEM((CHUNK,), _i32),
            pltpu.VMEM((CHUNK, HALF), _i32),
            pltpu.VMEM((CHUNK, HALF), _i32),
            pltpu.VMEM((CHUNK,), _i32),
            pltpu.VMEM((CHUNK,), _i32),
            pltpu.VMEM((CHUNK, HALF), _i32),
            pltpu.VMEM((CHUNK, HALF), _i32),
            pltpu.SemaphoreType.DMA,
            pltpu.SemaphoreType.DMA,
            pltpu.SemaphoreType.DMA,
            pltpu.SemaphoreType.DMA,
        ],
        compiler_params=pltpu.CompilerParams(needs_layout_passes=False),
    )


_sc_gather0 = _make_gather(0)
_sc_gather1 = _make_gather(EHALF)


def _sc_scatter_body(off, m_hbm, sidx_hbm,
                     agg_hbm,
                     idx0, mb0, idx1, mb1, zbuf, acc_sh, semm0, semm1):
    c = lax.axis_index("c")
    s = lax.axis_index("s")

    def zrow(r, carry):
        for g in range(HALF // 16):
            zbuf[r, pl.ds(g * 16, 16)] = jnp.zeros((16,), _f32)
        return carry

    lax.fori_loop(0, 64, zrow, 0)
    rows_per_sub = ACC_ROWS // NS
    for k in range(rows_per_sub // 64):
        pltpu.sync_copy(zbuf, acc_sh.at[pl.ds(s * rows_per_sub + k * 64, 64)])
    plsc.subcore_barrier()

    e_per_sub = EHALF // NS
    n_chunks = e_per_sub // CHUNK
    sbase = s * e_per_sub

    slots = ((idx0, mb0, semm0), (idx1, mb1, semm1))

    def startm(j, b):
        idx, mb, semm = slots[b]
        base = sbase + j * CHUNK
        pltpu.sync_copy(sidx_hbm.at[pl.ds(off + base, CHUNK)], idx)
        pltpu.async_copy(m_hbm.at[c, pl.ds(base, CHUNK)], mb, semm)

    def finishm(j, b):
        idx, mb, semm = slots[b]
        base = sbase + j * CHUNK
        pltpu.make_async_copy(m_hbm.at[c, pl.ds(base, CHUNK)], mb, semm).wait()
        pltpu.sync_copy(mb, acc_sh.at[idx], add=True)

    startm(0, 0)
    startm(1, 1)

    def pair(jj, carry):
        j0 = 2 * jj
        finishm(j0, 0)
        startm(j0 + 2, 0)
        finishm(j0 + 1, 1)
        startm(j0 + 3, 1)
        return carry

    lax.fori_loop(0, n_chunks // 2 - 1, pair, 0)
    finishm(n_chunks - 2, 0)
    finishm(n_chunks - 1, 1)
    plsc.subcore_barrier()

    out_per_sub = NP // NS
    pltpu.sync_copy(acc_sh.at[pl.ds(s * out_per_sub, out_per_sub)],
                    agg_hbm.at[c, pl.ds(s * out_per_sub, out_per_sub)])


def _make_scatter(off):
    return pl.kernel(
        functools.partial(_sc_scatter_body, off),
        out_type=jax.ShapeDtypeStruct((NC, NP, HALF), _f32),
        mesh=_SC_MESH,
        scratch_types=[
            pltpu.VMEM((CHUNK,), _i32),
            pltpu.VMEM((CHUNK, HALF), _f32),
            pltpu.VMEM((CHUNK,), _i32),
            pltpu.VMEM((CHUNK, HALF), _f32),
            pltpu.VMEM((64, HALF), _f32),
            pltpu.VMEM_SHARED((ACC_ROWS, HALF), _f32),
            pltpu.SemaphoreType.DMA,
            pltpu.SemaphoreType.DMA,
        ],
    )


_sc_scatter0 = _make_scatter(0)
_sc_scatter1 = _make_scatter(EHALF)


EBLK = 512
NBLK = 256


def _pack_tc(a):
    lo = lax.bitcast_convert_type(
        a[:, :HALF].astype(_bf16).astype(_f32), _i32)
    hi = lax.bitcast_convert_type(
        a[:, HALF:].astype(_bf16).astype(_f32), _i32)
    return jnp.bitwise_or(lax.shift_right_logical(lo, jnp.int32(16)),
                          jnp.bitwise_and(hi, jnp.int32(-65536)))


def _unpack_tc(w):
    lo = lax.bitcast_convert_type(lax.shift_left(w, jnp.int32(16)), _f32)
    hi = lax.bitcast_convert_type(jnp.bitwise_and(w, jnp.int32(-65536)), _f32)
    return jnp.concatenate([lo, hi], axis=1)


def _tc_pq_body(x_ref, wa_ref, wb_ref, p_ref, q_ref):
    x = x_ref[...]
    p_ref[...] = _pack_tc(jnp.dot(x, wa_ref[...], preferred_element_type=_f32))
    q_ref[...] = _pack_tc(jnp.dot(x, wb_ref[...], preferred_element_type=_f32))


def _tc_pq(x, wa, wb):
    return pl.pallas_call(
        _tc_pq_body,
        grid=(NP // NBLK,),
        in_specs=[
            pl.BlockSpec((NBLK, DIM), lambda i: (i, 0)),
            pl.BlockSpec((DIM, DIM), lambda i: (0, 0)),
            pl.BlockSpec((DIM, DIM), lambda i: (0, 0)),
        ],
        out_specs=[
            pl.BlockSpec((NBLK, HALF), lambda i: (i, 0)),
            pl.BlockSpec((NBLK, HALF), lambda i: (i, 0)),
        ],
        out_shape=[
            jax.ShapeDtypeStruct((NP, HALF), _i32),
            jax.ShapeDtypeStruct((NP, HALF), _i32),
        ],
    )(x, wa, wb)


def _bessel_block(dd, freq):
    dist = jnp.sqrt(dd + 1e-12)
    ds = dist / CUTOFF
    p = 6.0
    a = -(p + 1.0) * (p + 2.0) / 2.0
    b = p * (p + 2.0)
    c = -p * (p + 1.0) / 2.0
    ds2 = ds * ds
    ds4 = ds2 * ds2
    ds5 = ds4 * ds
    ds6 = ds5 * ds
    ds7 = ds6 * ds
    env = (1.0 / ds + a * ds5 + b * ds6 + c * ds7) * (ds < 1.0).astype(_f32)
    return env * jnp.sin(freq * ds)


def _tc_edge_body(s_ref, dd_ref, freq_ref, w1c_ref, b1_ref,
                  w2_ref, b2_ref, w3_ref, b3_ref, m_ref):
    ea = _bessel_block(dd_ref[...], freq_ref[...])
    h = _unpack_tc(s_ref[...]) + lax.dot_general(
        ea, w1c_ref[...], (((1,), (0,)), ((), ())),
        preferred_element_type=_f32) + b1_ref[...]
    h = _silu(h)
    h = _silu(jnp.dot(h, w2_ref[...], preferred_element_type=_f32) + b2_ref[...])
    m = jnp.dot(h, w3_ref[...], preferred_element_type=_f32) + b3_ref[...]
    m_ref[0] = m[:, :HALF]
    m_ref[1] = m[:, HALF:]


def _tc_edge(s, dd2, freq2, w1c, b1, w2, b2, w3, b3, half):
    nblk = EHALF // EBLK
    off = half * nblk
    return pl.pallas_call(
        _tc_edge_body,
        grid=(nblk,),
        in_specs=[
            pl.BlockSpec((EBLK, HALF), lambda i: (i, 0)),
            pl.BlockSpec((EBLK, 1), lambda i: (i + off, 0)),
            pl.BlockSpec((1, NUM_BASIS), lambda i: (0, 0)),
            pl.BlockSpec((NUM_BASIS, DIM), lambda i: (0, 0)),
            pl.BlockSpec((1, DIM), lambda i: (0, 0)),
            pl.BlockSpec((DIM, DIM), lambda i: (0, 0)),
            pl.BlockSpec((1, DIM), lambda i: (0, 0)),
            pl.BlockSpec((DIM, DIM), lambda i: (0, 0)),
            pl.BlockSpec((1, DIM), lambda i: (0, 0)),
        ],
        out_specs=pl.BlockSpec((NC, EBLK, HALF), lambda i: (0, i, 0)),
        out_shape=jax.ShapeDtypeStruct((NC, EHALF, HALF), _f32),
    )(s, dd2, freq2, w1c, b1, w2, b2, w3, b3)


def _tc_upd_body(emit_pq, x_ref, agg1_ref, agg2_ref, u1x_ref, u1g_ref,
                 ub1_ref, u2_ref, ub2_ref, u3_ref, ub3_ref, *rest):
    if emit_pq:
        wa_ref, wb_ref, xn_ref, p_ref, q_ref = rest
    else:
        (xn_ref,) = rest
    x = x_ref[...]
    aggc = jnp.concatenate([agg1_ref[0] + agg2_ref[0],
                            agg1_ref[1] + agg2_ref[1]], axis=1)
    h = (jnp.dot(x, u1x_ref[...], preferred_element_type=_f32)
         + jnp.dot(aggc, u1g_ref[...], preferred_element_type=_f32)
         + ub1_ref[...])
    h = _silu(h)
    h = _silu(jnp.dot(h, u2_ref[...], preferred_element_type=_f32) + ub2_ref[...])
    xn = x + jnp.dot(h, u3_ref[...], preferred_element_type=_f32) + ub3_ref[...]
    xn_ref[...] = xn
    if emit_pq:
        p_ref[...] = _pack_tc(jnp.dot(xn, wa_ref[...],
                                      preferred_element_type=_f32))
        q_ref[...] = _pack_tc(jnp.dot(xn, wb_ref[...],
                                      preferred_element_type=_f32))


def _tc_upd(x, agg1, agg2, u1x, u1g, ub1, u2, ub2, u3, ub3, wa=None, wb=None):
    emit_pq = wa is not None
    full = lambda i: (0, 0)
    in_specs = [
        pl.BlockSpec((NBLK, DIM), lambda i: (i, 0)),
        pl.BlockSpec((NC, NBLK, HALF), lambda i: (0, i, 0)),
        pl.BlockSpec((NC, NBLK, HALF), lambda i: (0, i, 0)),
        pl.BlockSpec((DIM, DIM), full),
        pl.BlockSpec((DIM, DIM), full),
        pl.BlockSpec((1, DIM), full),
        pl.BlockSpec((DIM, DIM), full),
        pl.BlockSpec((1, DIM), full),
        pl.BlockSpec((DIM, DIM), full),
        pl.BlockSpec((1, DIM), full),
    ]
    args = [x, agg1, agg2, u1x, u1g, ub1, u2, ub2, u3, ub3]
    nblk = pl.BlockSpec((NBLK, DIM), lambda i: (i, 0))
    hblk = pl.BlockSpec((NBLK, HALF), lambda i: (i, 0))
    nshape = jax.ShapeDtypeStruct((NP, DIM), _f32)
    hshape = jax.ShapeDtypeStruct((NP, HALF), _i32)
    if emit_pq:
        in_specs += [pl.BlockSpec((DIM, DIM), full), pl.BlockSpec((DIM, DIM), full)]
        args += [wa, wb]
        out_specs = [nblk, hblk, hblk]
        out_shape = [nshape, hshape, hshape]
    else:
        out_specs = [nblk]
        out_shape = [nshape]
    return pl.pallas_call(
        functools.partial(_tc_upd_body, emit_pq),
        grid=(NP // NBLK,),
        in_specs=in_specs,
        out_specs=out_specs,
        out_shape=out_shape,
    )(*args)


def _tc_tail_body(x_ref, batch_ref, h1_ref, hb1_ref, h2_ref, hb2_ref,
                  h3_ref, hb3_ref, out_ref, acc_s, cnt_s):
    i = pl.program_id(0)

    @pl.when(i == 0)
    def _init():
        acc_s[...] = jnp.zeros_like(acc_s)
        cnt_s[...] = jnp.zeros_like(cnt_s)

    gids = lax.broadcasted_iota(_i32, (1, N_GRAPHS), 1)
    oh = (batch_ref[...] == gids).astype(_f32)
    x = x_ref[...]
    acc_s[...] += lax.dot_general(oh, x, (((0,), (0,)), ((), ())),
                                  preferred_element_type=_f32)
    ones = jnp.ones((NBLK, 1), _f32)
    cnt_s[...] += lax.dot_general(oh, ones, (((0,), (0,)), ((), ())),
                                  preferred_element_type=_f32)

    @pl.when(i == pl.num_programs(0) - 1)
    def _final():
        pooled = acc_s[...] / jnp.maximum(cnt_s[...], 1.0)
        h = _silu(jnp.dot(pooled, h1_ref[...], preferred_element_type=_f32)
                  + hb1_ref[...])
        h = _silu(jnp.dot(h, h2_ref[...], preferred_element_type=_f32)
                  + hb2_ref[...])
        out_ref[...] = (jnp.dot(h, h3_ref[...], preferred_element_type=_f32)
                        + hb3_ref[...])


def _tc_tail(x, batch2, h1, hb1, h2, hb2, h3, hb3):
    full = lambda i: (0, 0)
    return pl.pallas_call(
        _tc_tail_body,
        grid=(NP // NBLK,),
        in_specs=[
            pl.BlockSpec((NBLK, DIM), lambda i: (i, 0)),
            pl.BlockSpec((NBLK, 1), lambda i: (i, 0)),
            pl.BlockSpec((DIM, DIM), full),
            pl.BlockSpec((1, DIM), full),
            pl.BlockSpec((DIM, DIM), full),
            pl.BlockSpec((1, DIM), full),
            pl.BlockSpec((DIM, 1), full),
            pl.BlockSpec((1, 1), full),
        ],
        out_specs=pl.BlockSpec((N_GRAPHS, 1), full),
        out_shape=jax.ShapeDtypeStruct((N_GRAPHS, 1), _f32),
        scratch_shapes=[
            pltpu.VMEM((N_GRAPHS, DIM), _f32),
            pltpu.VMEM((N_GRAPHS, 1), _f32),
        ],
    )(x, batch2, h1, hb1, h2, hb2, h3, hb3)


def kernel(z, edge_index, batch, pos, emb, freq, layers, head):
    row = edge_index[0].astype(_i32)
    col = edge_index[1].astype(_i32)
    rowp = jnp.pad(row, (0, EP - N_EDGES))
    colp = jnp.pad(col, (0, EP - N_EDGES))
    sidx = jnp.pad(col, (0, EP - N_EDGES), constant_values=N_NODES)
    zp = jnp.pad(z.astype(_i32), (0, NP - N_NODES))
    batchp = jnp.pad(batch.astype(_i32), (0, NP - N_NODES),
                     constant_values=N_GRAPHS).reshape(NP, 1)
    posT = pos.T.reshape(3 * N_NODES)
    freq2 = freq.reshape(1, NUM_BASIS)

    x0, dd = _sc_prep(posT, zp, rowp, colp, emb)
    dd2 = dd.reshape(EP, 1)

    def msg_parts(layer):
        (w1, b1), (w2, b2), (w3, b3) = layer["msg"]
        return (w1[:DIM], w1[DIM:2 * DIM], w1[2 * DIM:],
                b1.reshape(1, DIM), w2, b2.reshape(1, DIM), w3,
                b3.reshape(1, DIM))

    def upd_parts(layer):
        (u1, ub1), (u2, ub2), (u3, ub3) = layer["upd"]
        return (u1[:DIM], u1[DIM:], ub1.reshape(1, DIM), u2,
                ub2.reshape(1, DIM), u3, ub3.reshape(1, DIM))

    x = x0
    wa0, wb0 = msg_parts(layers[0])[0], msg_parts(layers[0])[1]
    p, q = _tc_pq(x0, wa0, wb0)
    for li, layer in enumerate(layers):
        _, _, w1c, b1, w2, b2, w3, b3 = msg_parts(layer)
        s1 = _sc_gather0(p, q, rowp, colp)
        m1 = _tc_edge(s1, dd2, freq2, w1c, b1, w2, b2, w3, b3, 0)
        s2 = _sc_gather1(p, q, rowp, colp)
        agg1 = _sc_scatter0(m1, sidx)
        m2 = _tc_edge(s2, dd2, freq2, w1c, b1, w2, b2, w3, b3, 1)
        agg2 = _sc_scatter1(m2, sidx)
        u1x, u1g, ub1, u2, ub2, u3, ub3 = upd_parts(layer)
        if li + 1 < len(layers):
            wa, wb = msg_parts(layers[li + 1])[0], msg_parts(layers[li + 1])[1]
            x, p, q = _tc_upd(x, agg1, agg2, u1x, u1g, ub1, u2, ub2, u3, ub3,
                              wa, wb)
        else:
            (x,) = _tc_upd(x, agg1, agg2, u1x, u1g, ub1, u2, ub2, u3, ub3)

    (h1, hb1), (h2, hb2), (h3, hb3) = head
    return _tc_tail(x, batchp, h1, hb1.reshape(1, DIM), h2,
                    hb2.reshape(1, DIM), h3, hb3.reshape(1, 1))


# speedup vs baseline: 1.1974x; 1.0547x over previous
import functools

import jax
import jax.numpy as jnp
from jax import lax
from jax.experimental import pallas as pl
from jax.experimental.pallas import tpu as pltpu
from jax.experimental.pallas import tpu_sc as plsc

N_NODES = 10000
N_EDGES = 160000
DIM = 256
HALF = 128
NUM_BASIS = 16
CUTOFF = 5.0
N_GRAPHS = 8

NC = 2
NS = 16
NW = NC * NS
NP = 10240
EP = 163840
CHUNK = 128
E_PER_W = EP // NW
N_CHUNKS = E_PER_W // CHUNK
N_PER_W = NP // NW
ACC_ROWS = NP
EHALF = EP // 2

_f32 = jnp.float32
_i32 = jnp.int32
_bf16 = jnp.bfloat16


def _silu(x):
    return x * jax.nn.sigmoid(x)


_SC_MESH = plsc.VectorSubcoreMesh(core_axis_name="c", subcore_axis_name="s")


def _sc_prep_body(posT_hbm, z_hbm, row_hbm, col_hbm, emb_hbm,
                  x0_hbm, dd_hbm,
                  posT_v, zidx_v, ridx_v, cidx_v, dd_v, rows_v, sem):
    c = lax.axis_index("c")
    s = lax.axis_index("s")
    w = c * NS + s

    pltpu.sync_copy(posT_hbm, posT_v)

    for j in range(5):
        base = w * N_PER_W + j * 64
        pltpu.sync_copy(z_hbm.at[pl.ds(base, 64)], zidx_v)
        pltpu.async_copy(emb_hbm.at[zidx_v], rows_v, sem).wait()
        pltpu.sync_copy(rows_v, x0_hbm.at[pl.ds(base, 64)])

    def dd_chunk(j, carry):
        base = w * E_PER_W + j * CHUNK
        pltpu.sync_copy(row_hbm.at[pl.ds(base, CHUNK)], ridx_v)
        pltpu.sync_copy(col_hbm.at[pl.ds(base, CHUNK)], cidx_v)
        for g in range(CHUNK // 16):
            r = ridx_v[pl.ds(g * 16, 16)]
            cc = cidx_v[pl.ds(g * 16, 16)]
            acc = jnp.zeros((16,), _f32)
            for d in range(3):
                off = jnp.full((16,), d * N_NODES, _i32)
                xr = plsc.load_gather(posT_v, [off + r])
                xc = plsc.load_gather(posT_v, [off + cc])
                df = xr - xc
                acc = acc + df * df
            dd_v[pl.ds(g * 16, 16)] = acc
        pltpu.sync_copy(dd_v, dd_hbm.at[pl.ds(base, CHUNK)])
        return carry

    lax.fori_loop(0, N_CHUNKS, dd_chunk, 0)


_sc_prep = pl.kernel(
    _sc_prep_body,
    out_type=[
        jax.ShapeDtypeStruct((NP, DIM), _f32),
        jax.ShapeDtypeStruct((EP,), _f32),
    ],
    mesh=_SC_MESH,
    scratch_types=[
        pltpu.VMEM((3 * N_NODES,), _f32),
        pltpu.VMEM((64,), _i32),
        pltpu.VMEM((CHUNK,), _i32),
        pltpu.VMEM((CHUNK,), _i32),
        pltpu.VMEM((CHUNK,), _f32),
        pltpu.VMEM((64, DIM), _f32),
        pltpu.SemaphoreType.DMA,
    ],
    compiler_params=pltpu.CompilerParams(needs_layout_passes=False),
)


EH_PER_W = EHALF // NW
NH_CHUNKS = EH_PER_W // CHUNK


def _sc_gather_body(off, p_hbm, q_hbm, row_hbm, col_hbm,
                    s_hbm,
                    cidx0, ridx0, bp0, bq0,
                    cidx1, ridx1, bp1, bq1,
                    semg0, semg1, semw0, semw1):
    c = lax.axis_index("c")
    s = lax.axis_index("s")
    w = c * NS + s
    wbase = w * EH_PER_W

    slots = ((cidx0, ridx0, bp0, bq0, semg0, semw0),
             (cidx1, ridx1, bp1, bq1, semg1, semw1))

    def start(j, b):
        cidx, ridx, bp, bq, semg, _ = slots[b]
        base = wbase + j * CHUNK
        pltpu.sync_copy(col_hbm.at[pl.ds(off + base, CHUNK)], cidx)
        pltpu.sync_copy(row_hbm.at[pl.ds(off + base, CHUNK)], ridx)
        pltpu.async_copy(p_hbm.at[cidx], bp, semg)
        pltpu.async_copy(q_hbm.at[ridx], bq, semg)

    def finish(j, b):
        cidx, ridx, bp, bq, semg, semw = slots[b]
        base = wbase + j * CHUNK
        pltpu.make_async_copy(p_hbm.at[cidx], bp, semg).wait()
        pltpu.make_async_copy(q_hbm.at[ridx], bq, semg).wait()

        def add_row(r, carry2):
            for g in range(HALF // 16):
                sl = pl.ds(g * 16, 16)
                a = plsc.bitcast(bp[r, sl], _bf16)
                b = plsc.bitcast(bq[r, sl], _bf16)
                bp[r, sl] = plsc.bitcast(a + b, _i32)
            return carry2

        lax.fori_loop(0, CHUNK, add_row, 0)
        pltpu.async_copy(bp, s_hbm.at[pl.ds(base, CHUNK)], semw)

    def wait_wb(j, b):
        _, _, bp, _, _, semw = slots[b]
        base = wbase + j * CHUNK
        pltpu.make_async_copy(bp, s_hbm.at[pl.ds(base, CHUNK)], semw).wait()

    start(0, 0)
    start(1, 1)

    def pair(jj, carry):
        j0 = 2 * jj
        finish(j0, 0)
        wait_wb(j0, 0)
        start(j0 + 2, 0)
        finish(j0 + 1, 1)
        wait_wb(j0 + 1, 1)
        start(j0 + 3, 1)
        return carry

    lax.fori_loop(0, NH_CHUNKS // 2 - 1, pair, 0)
    finish(NH_CHUNKS - 2, 0)
    finish(NH_CHUNKS - 1, 1)
    wait_wb(NH_CHUNKS - 2, 0)
    wait_wb(NH_CHUNKS - 1, 1)


def _make_gather(off):
    return pl.kernel(
        functools.partial(_sc_gather_body, off),
        out_type=jax.ShapeDtypeStruct((EHALF, HALF), _i32),
        mesh=_SC_MESH,
        scratch_types=[
            pltpu.VMEM((CHUNK,), _i32),
            pltpu.VMEM((CHUNK,), _i32),
            pltpu.VMEM((CHUNK, HALF), _i32),
            pltpu.VMEM((CHUNK, HALF), _i32),
            pltpu.VMEM((CHUNK,), _i32),
            pltpu.VMEM((CHUNK,), _i32),
            pltpu.VMEM((CHUNK, HALF), _i32),
            pltpu.VMEM((CHUNK, HALF), _i32),
            pltpu.SemaphoreType.DMA,
            pltpu.SemaphoreType.DMA,
            pltpu.SemaphoreType.DMA,
            pltpu.SemaphoreType.DMA,
        ],
        compiler_params=pltpu.CompilerParams(needs_layout_passes=False),
    )


_sc_gather0 = _make_gather(0)
_sc_gather1 = _make_gather(EHALF)


def _sc_scatter_body(off, m_hbm, sidx_hbm,
                     agg_hbm,
                     idx0, mb0, idx1, mb1, zbuf, acc_sh, semm0, semm1):
    c = lax.axis_index("c")
    s = lax.axis_index("s")

    def zrow(r, carry):
        for g in range(HALF // 16):
            zbuf[r, pl.ds(g * 16, 16)] = jnp.zeros((16,), _f32)
        return carry

    lax.fori_loop(0, 64, zrow, 0)
    rows_per_sub = ACC_ROWS // NS
    for k in range(rows_per_sub // 64):
        pltpu.sync_copy(zbuf, acc_sh.at[pl.ds(s * rows_per_sub + k * 64, 64)])
    plsc.subcore_barrier()

    e_per_sub = EHALF // NS
    n_chunks = e_per_sub // CHUNK
    sbase = s * e_per_sub

    slots = ((idx0, mb0, semm0), (idx1, mb1, semm1))

    def startm(j, b):
        idx, mb, semm = slots[b]
        base = sbase + j * CHUNK
        pltpu.sync_copy(sidx_hbm.at[pl.ds(off + base, CHUNK)], idx)
        pltpu.async_copy(m_hbm.at[c, pl.ds(base, CHUNK)], mb, semm)

    def finishm(j, b):
        idx, mb, semm = slots[b]
        base = sbase + j * CHUNK
        pltpu.make_async_copy(m_hbm.at[c, pl.ds(base, CHUNK)], mb, semm).wait()
        pltpu.sync_copy(mb, acc_sh.at[idx], add=True)

    startm(0, 0)
    startm(1, 1)

    def pair(jj, carry):
        j0 = 2 * jj
        finishm(j0, 0)
        startm(j0 + 2, 0)
        finishm(j0 + 1, 1)
        startm(j0 + 3, 1)
        return carry

    lax.fori_loop(0, n_chunks // 2 - 1, pair, 0)
    finishm(n_chunks - 2, 0)
    finishm(n_chunks - 1, 1)
    plsc.subcore_barrier()

    out_per_sub = NP // NS
    pltpu.sync_copy(acc_sh.at[pl.ds(s * out_per_sub, out_per_sub)],
                    agg_hbm.at[c, pl.ds(s * out_per_sub, out_per_sub)])


def _make_scatter(off):
    return pl.kernel(
        functools.partial(_sc_scatter_body, off),
        out_type=jax.ShapeDtypeStruct((NC, NP, HALF), _f32),
        mesh=_SC_MESH,
        scratch_types=[
            pltpu.VMEM((CHUNK,), _i32),
            pltpu.VMEM((CHUNK, HALF), _f32),
            pltpu.VMEM((CHUNK,), _i32),
            pltpu.VMEM((CHUNK, HALF), _f32),
            pltpu.VMEM((64, HALF), _f32),
            pltpu.VMEM_SHARED((ACC_ROWS, HALF), _f32),
            pltpu.SemaphoreType.DMA,
            pltpu.SemaphoreType.DMA,
        ],
    )


_sc_scatter0 = _make_scatter(0)
_sc_scatter1 = _make_scatter(EHALF)


EBLK = 1024
NBLK = 256


def _pack_tc(a):
    lo = lax.bitcast_convert_type(
        a[:, :HALF].astype(_bf16).astype(_f32), _i32)
    hi = lax.bitcast_convert_type(
        a[:, HALF:].astype(_bf16).astype(_f32), _i32)
    return jnp.bitwise_or(lax.shift_right_logical(lo, jnp.int32(16)),
                          jnp.bitwise_and(hi, jnp.int32(-65536)))


def _unpack_tc(w):
    lo = lax.bitcast_convert_type(lax.shift_left(w, jnp.int32(16)), _f32)
    hi = lax.bitcast_convert_type(jnp.bitwise_and(w, jnp.int32(-65536)), _f32)
    return jnp.concatenate([lo, hi], axis=1)


def _tc_pq_body(x_ref, wa_ref, wb_ref, p_ref, q_ref):
    x = x_ref[...]
    p_ref[...] = _pack_tc(jnp.dot(x, wa_ref[...], preferred_element_type=_f32))
    q_ref[...] = _pack_tc(jnp.dot(x, wb_ref[...], preferred_element_type=_f32))


def _tc_pq(x, wa, wb):
    return pl.pallas_call(
        _tc_pq_body,
        grid=(NP // NBLK,),
        in_specs=[
            pl.BlockSpec((NBLK, DIM), lambda i: (i, 0)),
            pl.BlockSpec((DIM, DIM), lambda i: (0, 0)),
            pl.BlockSpec((DIM, DIM), lambda i: (0, 0)),
        ],
        out_specs=[
            pl.BlockSpec((NBLK, HALF), lambda i: (i, 0)),
            pl.BlockSpec((NBLK, HALF), lambda i: (i, 0)),
        ],
        out_shape=[
            jax.ShapeDtypeStruct((NP, HALF), _i32),
            jax.ShapeDtypeStruct((NP, HALF), _i32),
        ],
    )(x, wa, wb)


def _bessel_block(dd, freq):
    dist = jnp.sqrt(dd + 1e-12)
    ds = dist / CUTOFF
    p = 6.0
    a = -(p + 1.0) * (p + 2.0) / 2.0
    b = p * (p + 2.0)
    c = -p * (p + 1.0) / 2.0
    ds2 = ds * ds
    ds4 = ds2 * ds2
    ds5 = ds4 * ds
    ds6 = ds5 * ds
    ds7 = ds6 * ds
    env = (1.0 / ds + a * ds5 + b * ds6 + c * ds7) * (ds < 1.0).astype(_f32)
    return env * jnp.sin(freq * ds)


def _tc_edge_body(s_ref, dd_ref, freq_ref, w1c_ref, b1_ref,
                  w2_ref, b2_ref, w3_ref, b3_ref, m_ref):
    ea = _bessel_block(dd_ref[...], freq_ref[...])
    h = _unpack_tc(s_ref[...]) + lax.dot_general(
        ea, w1c_ref[...], (((1,), (0,)), ((), ())),
        preferred_element_type=_f32) + b1_ref[...]
    h = _silu(h)
    h = _silu(jnp.dot(h, w2_ref[...], preferred_element_type=_f32) + b2_ref[...])
    m = jnp.dot(h, w3_ref[...], preferred_element_type=_f32) + b3_ref[...]
    m_ref[0] = m[:, :HALF]
    m_ref[1] = m[:, HALF:]


def _tc_edge(s, dd2, freq2, w1c, b1, w2, b2, w3, b3, half):
    nblk = EHALF // EBLK
    off = half * nblk
    return pl.pallas_call(
        _tc_edge_body,
        grid=(nblk,),
        in_specs=[
            pl.BlockSpec((EBLK, HALF), lambda i: (i, 0)),
            pl.BlockSpec((EBLK, 1), lambda i: (i + off, 0)),
            pl.BlockSpec((1, NUM_BASIS), lambda i: (0, 0)),
            pl.BlockSpec((NUM_BASIS, DIM), lambda i: (0, 0)),
            pl.BlockSpec((1, DIM), lambda i: (0, 0)),
            pl.BlockSpec((DIM, DIM), lambda i: (0, 0)),
            pl.BlockSpec((1, DIM), lambda i: (0, 0)),
            pl.BlockSpec((DIM, DIM), lambda i: (0, 0)),
            pl.BlockSpec((1, DIM), lambda i: (0, 0)),
        ],
        out_specs=pl.BlockSpec((NC, EBLK, HALF), lambda i: (0, i, 0)),
        out_shape=jax.ShapeDtypeStruct((NC, EHALF, HALF), _f32),
    )(s, dd2, freq2, w1c, b1, w2, b2, w3, b3)


def _tc_upd_body(emit_pq, x_ref, agg1_ref, agg2_ref, u1x_ref, u1g_ref,
                 ub1_ref, u2_ref, ub2_ref, u3_ref, ub3_ref, *rest):
    if emit_pq:
        wa_ref, wb_ref, xn_ref, p_ref, q_ref = rest
    else:
        (xn_ref,) = rest
    x = x_ref[...]
    aggc = jnp.concatenate([agg1_ref[0] + agg2_ref[0],
                            agg1_ref[1] + agg2_ref[1]], axis=1)
    h = (jnp.dot(x, u1x_ref[...], preferred_element_type=_f32)
         + jnp.dot(aggc, u1g_ref[...], preferred_element_type=_f32)
         + ub1_ref[...])
    h = _silu(h)
    h = _silu(jnp.dot(h, u2_ref[...], preferred_element_type=_f32) + ub2_ref[...])
    xn = x + jnp.dot(h, u3_ref[...], preferred_element_type=_f32) + ub3_ref[...]
    xn_ref[...] = xn
    if emit_pq:
        p_ref[...] = _pack_tc(jnp.dot(xn, wa_ref[...],
                                      preferred_element_type=_f32))
        q_ref[...] = _pack_tc(jnp.dot(xn, wb_ref[...],
                                      preferred_element_type=_f32))


def _tc_upd(x, agg1, agg2, u1x, u1g, ub1, u2, ub2, u3, ub3, wa=None, wb=None):
    emit_pq = wa is not None
    full = lambda i: (0, 0)
    in_specs = [
        pl.BlockSpec((NBLK, DIM), lambda i: (i, 0)),
        pl.BlockSpec((NC, NBLK, HALF), lambda i: (0, i, 0)),
        pl.BlockSpec((NC, NBLK, HALF), lambda i: (0, i, 0)),
        pl.BlockSpec((DIM, DIM), full),
        pl.BlockSpec((DIM, DIM), full),
        pl.BlockSpec((1, DIM), full),
        pl.BlockSpec((DIM, DIM), full),
        pl.BlockSpec((1, DIM), full),
        pl.BlockSpec((DIM, DIM), full),
        pl.BlockSpec((1, DIM), full),
    ]
    args = [x, agg1, agg2, u1x, u1g, ub1, u2, ub2, u3, ub3]
    nblk = pl.BlockSpec((NBLK, DIM), lambda i: (i, 0))
    hblk = pl.BlockSpec((NBLK, HALF), lambda i: (i, 0))
    nshape = jax.ShapeDtypeStruct((NP, DIM), _f32)
    hshape = jax.ShapeDtypeStruct((NP, HALF), _i32)
    if emit_pq:
        in_specs += [pl.BlockSpec((DIM, DIM), full), pl.BlockSpec((DIM, DIM), full)]
        args += [wa, wb]
        out_specs = [nblk, hblk, hblk]
        out_shape = [nshape, hshape, hshape]
    else:
        out_specs = [nblk]
        out_shape = [nshape]
    return pl.pallas_call(
        functools.partial(_tc_upd_body, emit_pq),
        grid=(NP // NBLK,),
        in_specs=in_specs,
        out_specs=out_specs,
        out_shape=out_shape,
    )(*args)


def _tc_tail_body(x_ref, batch_ref, h1_ref, hb1_ref, h2_ref, hb2_ref,
                  h3_ref, hb3_ref, out_ref, acc_s, cnt_s):
    i = pl.program_id(0)

    @pl.when(i == 0)
    def _init():
        acc_s[...] = jnp.zeros_like(acc_s)
        cnt_s[...] = jnp.zeros_like(cnt_s)

    gids = lax.broadcasted_iota(_i32, (1, N_GRAPHS), 1)
    oh = (batch_ref[...] == gids).astype(_f32)
    x = x_ref[...]
    acc_s[...] += lax.dot_general(oh, x, (((0,), (0,)), ((), ())),
                                  preferred_element_type=_f32)
    ones = jnp.ones((NBLK, 1), _f32)
    cnt_s[...] += lax.dot_general(oh, ones, (((0,), (0,)), ((), ())),
                                  preferred_element_type=_f32)

    @pl.when(i == pl.num_programs(0) - 1)
    def _final():
        pooled = acc_s[...] / jnp.maximum(cnt_s[...], 1.0)
        h = _silu(jnp.dot(pooled, h1_ref[...], preferred_element_type=_f32)
                  + hb1_ref[...])
        h = _silu(jnp.dot(h, h2_ref[...], preferred_element_type=_f32)
                  + hb2_ref[...])
        out_ref[...] = (jnp.dot(h, h3_ref[...], preferred_element_type=_f32)
                        + hb3_ref[...])


def _tc_tail(x, batch2, h1, hb1, h2, hb2, h3, hb3):
    full = lambda i: (0, 0)
    return pl.pallas_call(
        _tc_tail_body,
        grid=(NP // NBLK,),
        in_specs=[
            pl.BlockSpec((NBLK, DIM), lambda i: (i, 0)),
            pl.BlockSpec((NBLK, 1), lambda i: (i, 0)),
            pl.BlockSpec((DIM, DIM), full),
            pl.BlockSpec((1, DIM), full),
            pl.BlockSpec((DIM, DIM), full),
            pl.BlockSpec((1, DIM), full),
            pl.BlockSpec((DIM, 1), full),
            pl.BlockSpec((1, 1), full),
        ],
        out_specs=pl.BlockSpec((N_GRAPHS, 1), full),
        out_shape=jax.ShapeDtypeStruct((N_GRAPHS, 1), _f32),
        scratch_shapes=[
            pltpu.VMEM((N_GRAPHS, DIM), _f32),
            pltpu.VMEM((N_GRAPHS, 1), _f32),
        ],
    )(x, batch2, h1, hb1, h2, hb2, h3, hb3)


def kernel(z, edge_index, batch, pos, emb, freq, layers, head):
    row = edge_index[0].astype(_i32)
    col = edge_index[1].astype(_i32)
    rowp = jnp.pad(row, (0, EP - N_EDGES))
    colp = jnp.pad(col, (0, EP - N_EDGES))
    sidx = jnp.pad(col, (0, EP - N_EDGES), constant_values=N_NODES)
    zp = jnp.pad(z.astype(_i32), (0, NP - N_NODES))
    batchp = jnp.pad(batch.astype(_i32), (0, NP - N_NODES),
                     constant_values=N_GRAPHS).reshape(NP, 1)
    posT = pos.T.reshape(3 * N_NODES)
    freq2 = freq.reshape(1, NUM_BASIS)

    x0, dd = _sc_prep(posT, zp, rowp, colp, emb)
    dd2 = dd.reshape(EP, 1)

    def msg_parts(layer):
        (w1, b1), (w2, b2), (w3, b3) = layer["msg"]
        return (w1[:DIM], w1[DIM:2 * DIM], w1[2 * DIM:],
                b1.reshape(1, DIM), w2, b2.reshape(1, DIM), w3,
                b3.reshape(1, DIM))

    def upd_parts(layer):
        (u1, ub1), (u2, ub2), (u3, ub3) = layer["upd"]
        return (u1[:DIM], u1[DIM:], ub1.reshape(1, DIM), u2,
                ub2.reshape(1, DIM), u3, ub3.reshape(1, DIM))

    x = x0
    wa0, wb0 = msg_parts(layers[0])[0], msg_parts(layers[0])[1]
    p, q = _tc_pq(x0, wa0, wb0)
    for li, layer in enumerate(layers):
        _, _, w1c, b1, w2, b2, w3, b3 = msg_parts(layer)
        s1 = _sc_gather0(p, q, rowp, colp)
        m1 = _tc_edge(s1, dd2, freq2, w1c, b1, w2, b2, w3, b3, 0)
        s2 = _sc_gather1(p, q, rowp, colp)
        agg1 = _sc_scatter0(m1, sidx)
        m2 = _tc_edge(s2, dd2, freq2, w1c, b1, w2, b2, w3, b3, 1)
        agg2 = _sc_scatter1(m2, sidx)
        u1x, u1g, ub1, u2, ub2, u3, ub3 = upd_parts(layer)
        if li + 1 < len(layers):
            wa, wb = msg_parts(layers[li + 1])[0], msg_parts(layers[li + 1])[1]
            x, p, q = _tc_upd(x, agg1, agg2, u1x, u1g, ub1, u2, ub2, u3, ub3,
                              wa, wb)
        else:
            (x,) = _tc_upd(x, agg1, agg2, u1x, u1g, ub1, u2, ub2, u3, ub3)

    (h1, hb1), (h2, hb2), (h3, hb3) = head
    return _tc_tail(x, batchp, h1, hb1.reshape(1, DIM), h2,
                    hb2.reshape(1, DIM), h3, hb3.reshape(1, 1))


# speedup vs baseline: 1.2573x; 1.0500x over previous
import functools

import jax
import jax.numpy as jnp
from jax import lax
from jax.experimental import pallas as pl
from jax.experimental.pallas import tpu as pltpu
from jax.experimental.pallas import tpu_sc as plsc

N_NODES = 10000
N_EDGES = 160000
DIM = 256
HALF = 128
NUM_BASIS = 16
CUTOFF = 5.0
N_GRAPHS = 8

NC = 2
NS = 16
NW = NC * NS
NP = 10240
EP = 163840
CHUNK = 128
E_PER_W = EP // NW
N_CHUNKS = E_PER_W // CHUNK
N_PER_W = NP // NW
ACC_ROWS = NP
EHALF = EP // 2

_f32 = jnp.float32
_i32 = jnp.int32
_bf16 = jnp.bfloat16


def _silu(x):
    return x * jax.nn.sigmoid(x)


_SC_MESH = plsc.VectorSubcoreMesh(core_axis_name="c", subcore_axis_name="s")


def _sc_prep_body(posT_hbm, z_hbm, row_hbm, col_hbm, emb_hbm,
                  x0_hbm, dd_hbm,
                  posT_v, zidx_v, ridx_v, cidx_v, dd_v, rows_v, sem):
    c = lax.axis_index("c")
    s = lax.axis_index("s")
    w = c * NS + s

    pltpu.sync_copy(posT_hbm, posT_v)

    for j in range(5):
        base = w * N_PER_W + j * 64
        pltpu.sync_copy(z_hbm.at[pl.ds(base, 64)], zidx_v)
        pltpu.async_copy(emb_hbm.at[zidx_v], rows_v, sem).wait()
        pltpu.sync_copy(rows_v, x0_hbm.at[pl.ds(base, 64)])

    def dd_chunk(j, carry):
        base = w * E_PER_W + j * CHUNK
        pltpu.sync_copy(row_hbm.at[pl.ds(base, CHUNK)], ridx_v)
        pltpu.sync_copy(col_hbm.at[pl.ds(base, CHUNK)], cidx_v)
        for g in range(CHUNK // 16):
            r = ridx_v[pl.ds(g * 16, 16)]
            cc = cidx_v[pl.ds(g * 16, 16)]
            acc = jnp.zeros((16,), _f32)
            for d in range(3):
                off = jnp.full((16,), d * N_NODES, _i32)
                xr = plsc.load_gather(posT_v, [off + r])
                xc = plsc.load_gather(posT_v, [off + cc])
                df = xr - xc
                acc = acc + df * df
            dd_v[pl.ds(g * 16, 16)] = acc
        pltpu.sync_copy(dd_v, dd_hbm.at[pl.ds(base, CHUNK)])
        return carry

    lax.fori_loop(0, N_CHUNKS, dd_chunk, 0)


_sc_prep = pl.kernel(
    _sc_prep_body,
    out_type=[
        jax.ShapeDtypeStruct((NP, DIM), _f32),
        jax.ShapeDtypeStruct((EP,), _f32),
    ],
    mesh=_SC_MESH,
    scratch_types=[
        pltpu.VMEM((3 * N_NODES,), _f32),
        pltpu.VMEM((64,), _i32),
        pltpu.VMEM((CHUNK,), _i32),
        pltpu.VMEM((CHUNK,), _i32),
        pltpu.VMEM((CHUNK,), _f32),
        pltpu.VMEM((64, DIM), _f32),
        pltpu.SemaphoreType.DMA,
    ],
    compiler_params=pltpu.CompilerParams(needs_layout_passes=False),
)


EH_PER_W = EHALF // NW
NH_CHUNKS = EH_PER_W // CHUNK


def _sc_gather_body(off, p_hbm, q_hbm, row_hbm, col_hbm,
                    s_hbm,
                    cidx0, ridx0, bp0, bq0,
                    cidx1, ridx1, bp1, bq1,
                    semg0, semg1, semw0, semw1):
    c = lax.axis_index("c")
    s = lax.axis_index("s")
    w = c * NS + s
    wbase = w * EH_PER_W

    slots = ((cidx0, ridx0, bp0, bq0, semg0, semw0),
             (cidx1, ridx1, bp1, bq1, semg1, semw1))

    def start(j, b):
        cidx, ridx, bp, bq, semg, _ = slots[b]
        base = wbase + j * CHUNK
        pltpu.sync_copy(col_hbm.at[pl.ds(off + base, CHUNK)], cidx)
        pltpu.sync_copy(row_hbm.at[pl.ds(off + base, CHUNK)], ridx)
        pltpu.async_copy(p_hbm.at[cidx], bp, semg)
        pltpu.async_copy(q_hbm.at[ridx], bq, semg)

    def finish(j, b):
        cidx, ridx, bp, bq, semg, semw = slots[b]
        base = wbase + j * CHUNK
        pltpu.make_async_copy(p_hbm.at[cidx], bp, semg).wait()
        pltpu.make_async_copy(q_hbm.at[ridx], bq, semg).wait()

        def add_row(r, carry2):
            for g in range(HALF // 16):
                sl = pl.ds(g * 16, 16)
                a = plsc.bitcast(bp[r, sl], _bf16)
                b = plsc.bitcast(bq[r, sl], _bf16)
                bp[r, sl] = plsc.bitcast(a + b, _i32)
            return carry2

        lax.fori_loop(0, CHUNK, add_row, 0)
        pltpu.async_copy(bp, s_hbm.at[pl.ds(base, CHUNK)], semw)

    def wait_wb(j, b):
        _, _, bp, _, _, semw = slots[b]
        base = wbase + j * CHUNK
        pltpu.make_async_copy(bp, s_hbm.at[pl.ds(base, CHUNK)], semw).wait()

    start(0, 0)
    start(1, 1)

    def pair(jj, carry):
        j0 = 2 * jj
        finish(j0, 0)
        wait_wb(j0, 0)
        start(j0 + 2, 0)
        finish(j0 + 1, 1)
        wait_wb(j0 + 1, 1)
        start(j0 + 3, 1)
        return carry

    lax.fori_loop(0, NH_CHUNKS // 2 - 1, pair, 0)
    finish(NH_CHUNKS - 2, 0)
    finish(NH_CHUNKS - 1, 1)
    wait_wb(NH_CHUNKS - 2, 0)
    wait_wb(NH_CHUNKS - 1, 1)


def _make_gather(off):
    return pl.kernel(
        functools.partial(_sc_gather_body, off),
        out_type=jax.ShapeDtypeStruct((EHALF, HALF), _i32),
        mesh=_SC_MESH,
        scratch_types=[
            pltpu.VMEM((CHUNK,), _i32),
            pltpu.VMEM((CHUNK,), _i32),
            pltpu.VMEM((CHUNK, HALF), _i32),
            pltpu.VMEM((CHUNK, HALF), _i32),
            pltpu.VMEM((CHUNK,), _i32),
            pltpu.VMEM((CHUNK,), _i32),
            pltpu.VMEM((CHUNK, HALF), _i32),
            pltpu.VMEM((CHUNK, HALF), _i32),
            pltpu.SemaphoreType.DMA,
            pltpu.SemaphoreType.DMA,
            pltpu.SemaphoreType.DMA,
            pltpu.SemaphoreType.DMA,
        ],
        compiler_params=pltpu.CompilerParams(needs_layout_passes=False),
    )


_sc_gather0 = _make_gather(0)
_sc_gather1 = _make_gather(EHALF)


def _sc_scatter_body(off, m_hbm, sidx_hbm,
                     agg_hbm,
                     idx0, mb0, idx1, mb1, zbuf, acc_sh, semm0, semm1):
    c = lax.axis_index("c")
    s = lax.axis_index("s")

    def zrow(r, carry):
        for g in range(HALF // 16):
            zbuf[r, pl.ds(g * 16, 16)] = jnp.zeros((16,), _f32)
        return carry

    lax.fori_loop(0, 64, zrow, 0)
    rows_per_sub = ACC_ROWS // NS
    for k in range(rows_per_sub // 64):
        pltpu.sync_copy(zbuf, acc_sh.at[pl.ds(s * rows_per_sub + k * 64, 64)])
    plsc.subcore_barrier()

    e_per_sub = EHALF // NS
    n_chunks = e_per_sub // CHUNK
    sbase = s * e_per_sub

    slots = ((idx0, mb0, semm0), (idx1, mb1, semm1))

    def startm(j, b):
        idx, mb, semm = slots[b]
        base = sbase + j * CHUNK
        pltpu.sync_copy(sidx_hbm.at[pl.ds(off + base, CHUNK)], idx)
        pltpu.async_copy(m_hbm.at[c, pl.ds(base, CHUNK)], mb, semm)

    def finishm(j, b):
        idx, mb, semm = slots[b]
        base = sbase + j * CHUNK
        pltpu.make_async_copy(m_hbm.at[c, pl.ds(base, CHUNK)], mb, semm).wait()
        pltpu.sync_copy(mb, acc_sh.at[idx], add=True)

    startm(0, 0)
    startm(1, 1)

    def pair(jj, carry):
        j0 = 2 * jj
        finishm(j0, 0)
        startm(j0 + 2, 0)
        finishm(j0 + 1, 1)
        startm(j0 + 3, 1)
        return carry

    lax.fori_loop(0, n_chunks // 2 - 1, pair, 0)
    finishm(n_chunks - 2, 0)
    finishm(n_chunks - 1, 1)
    plsc.subcore_barrier()

    out_per_sub = NP // NS
    pltpu.sync_copy(acc_sh.at[pl.ds(s * out_per_sub, out_per_sub)],
                    agg_hbm.at[c, pl.ds(s * out_per_sub, out_per_sub)])


def _make_scatter(off):
    return pl.kernel(
        functools.partial(_sc_scatter_body, off),
        out_type=jax.ShapeDtypeStruct((NC, NP, HALF), _f32),
        mesh=_SC_MESH,
        scratch_types=[
            pltpu.VMEM((CHUNK,), _i32),
            pltpu.VMEM((CHUNK, HALF), _f32),
            pltpu.VMEM((CHUNK,), _i32),
            pltpu.VMEM((CHUNK, HALF), _f32),
            pltpu.VMEM((64, HALF), _f32),
            pltpu.VMEM_SHARED((ACC_ROWS, HALF), _f32),
            pltpu.SemaphoreType.DMA,
            pltpu.SemaphoreType.DMA,
        ],
    )


_sc_scatter0 = _make_scatter(0)
_sc_scatter1 = _make_scatter(EHALF)


EBLK = 2048
NBLK = 512


def _pack_tc(a):
    lo = lax.bitcast_convert_type(
        a[:, :HALF].astype(_bf16).astype(_f32), _i32)
    hi = lax.bitcast_convert_type(
        a[:, HALF:].astype(_bf16).astype(_f32), _i32)
    return jnp.bitwise_or(lax.shift_right_logical(lo, jnp.int32(16)),
                          jnp.bitwise_and(hi, jnp.int32(-65536)))


def _unpack_tc(w):
    lo = lax.bitcast_convert_type(lax.shift_left(w, jnp.int32(16)), _f32)
    hi = lax.bitcast_convert_type(jnp.bitwise_and(w, jnp.int32(-65536)), _f32)
    return jnp.concatenate([lo, hi], axis=1)


def _tc_pq_body(x_ref, wa_ref, wb_ref, p_ref, q_ref):
    x = x_ref[...]
    p_ref[...] = _pack_tc(jnp.dot(x, wa_ref[...], preferred_element_type=_f32))
    q_ref[...] = _pack_tc(jnp.dot(x, wb_ref[...], preferred_element_type=_f32))


def _tc_pq(x, wa, wb):
    return pl.pallas_call(
        _tc_pq_body,
        grid=(NP // NBLK,),
        in_specs=[
            pl.BlockSpec((NBLK, DIM), lambda i: (i, 0)),
            pl.BlockSpec((DIM, DIM), lambda i: (0, 0)),
            pl.BlockSpec((DIM, DIM), lambda i: (0, 0)),
        ],
        out_specs=[
            pl.BlockSpec((NBLK, HALF), lambda i: (i, 0)),
            pl.BlockSpec((NBLK, HALF), lambda i: (i, 0)),
        ],
        out_shape=[
            jax.ShapeDtypeStruct((NP, HALF), _i32),
            jax.ShapeDtypeStruct((NP, HALF), _i32),
        ],
    )(x, wa, wb)


def _bessel_block(dd, freq):
    dist = jnp.sqrt(dd + 1e-12)
    ds = dist / CUTOFF
    p = 6.0
    a = -(p + 1.0) * (p + 2.0) / 2.0
    b = p * (p + 2.0)
    c = -p * (p + 1.0) / 2.0
    ds2 = ds * ds
    ds4 = ds2 * ds2
    ds5 = ds4 * ds
    ds6 = ds5 * ds
    ds7 = ds6 * ds
    env = (1.0 / ds + a * ds5 + b * ds6 + c * ds7) * (ds < 1.0).astype(_f32)
    return env * jnp.sin(freq * ds)


def _tc_edge_body(s_ref, dd_ref, freq_ref, w1c_ref, b1_ref,
                  w2_ref, b2_ref, w3_ref, b3_ref, m_ref):
    ea = _bessel_block(dd_ref[...], freq_ref[...])
    h = _unpack_tc(s_ref[...]) + lax.dot_general(
        ea, w1c_ref[...], (((1,), (0,)), ((), ())),
        preferred_element_type=_f32) + b1_ref[...]
    h = _silu(h)
    h = _silu(jnp.dot(h, w2_ref[...], preferred_element_type=_f32) + b2_ref[...])
    m = jnp.dot(h, w3_ref[...], preferred_element_type=_f32) + b3_ref[...]
    m_ref[0] = m[:, :HALF]
    m_ref[1] = m[:, HALF:]


def _tc_edge(s, dd2, freq2, w1c, b1, w2, b2, w3, b3, half):
    nblk = EHALF // EBLK
    off = half * nblk
    return pl.pallas_call(
        _tc_edge_body,
        grid=(nblk,),
        in_specs=[
            pl.BlockSpec((EBLK, HALF), lambda i: (i, 0)),
            pl.BlockSpec((EBLK, 1), lambda i: (i + off, 0)),
            pl.BlockSpec((1, NUM_BASIS), lambda i: (0, 0)),
            pl.BlockSpec((NUM_BASIS, DIM), lambda i: (0, 0)),
            pl.BlockSpec((1, DIM), lambda i: (0, 0)),
            pl.BlockSpec((DIM, DIM), lambda i: (0, 0)),
            pl.BlockSpec((1, DIM), lambda i: (0, 0)),
            pl.BlockSpec((DIM, DIM), lambda i: (0, 0)),
            pl.BlockSpec((1, DIM), lambda i: (0, 0)),
        ],
        out_specs=pl.BlockSpec((NC, EBLK, HALF), lambda i: (0, i, 0)),
        out_shape=jax.ShapeDtypeStruct((NC, EHALF, HALF), _f32),
    )(s, dd2, freq2, w1c, b1, w2, b2, w3, b3)


def _tc_upd_body(emit_pq, x_ref, agg1_ref, agg2_ref, u1x_ref, u1g_ref,
                 ub1_ref, u2_ref, ub2_ref, u3_ref, ub3_ref, *rest):
    if emit_pq:
        wa_ref, wb_ref, xn_ref, p_ref, q_ref = rest
    else:
        (xn_ref,) = rest
    x = x_ref[...]
    aggc = jnp.concatenate([agg1_ref[0] + agg2_ref[0],
                            agg1_ref[1] + agg2_ref[1]], axis=1)
    h = (jnp.dot(x, u1x_ref[...], preferred_element_type=_f32)
         + jnp.dot(aggc, u1g_ref[...], preferred_element_type=_f32)
         + ub1_ref[...])
    h = _silu(h)
    h = _silu(jnp.dot(h, u2_ref[...], preferred_element_type=_f32) + ub2_ref[...])
    xn = x + jnp.dot(h, u3_ref[...], preferred_element_type=_f32) + ub3_ref[...]
    xn_ref[...] = xn
    if emit_pq:
        p_ref[...] = _pack_tc(jnp.dot(xn, wa_ref[...],
                                      preferred_element_type=_f32))
        q_ref[...] = _pack_tc(jnp.dot(xn, wb_ref[...],
                                      preferred_element_type=_f32))


def _tc_upd(x, agg1, agg2, u1x, u1g, ub1, u2, ub2, u3, ub3, wa=None, wb=None):
    emit_pq = wa is not None
    full = lambda i: (0, 0)
    in_specs = [
        pl.BlockSpec((NBLK, DIM), lambda i: (i, 0)),
        pl.BlockSpec((NC, NBLK, HALF), lambda i: (0, i, 0)),
        pl.BlockSpec((NC, NBLK, HALF), lambda i: (0, i, 0)),
        pl.BlockSpec((DIM, DIM), full),
        pl.BlockSpec((DIM, DIM), full),
        pl.BlockSpec((1, DIM), full),
        pl.BlockSpec((DIM, DIM), full),
        pl.BlockSpec((1, DIM), full),
        pl.BlockSpec((DIM, DIM), full),
        pl.BlockSpec((1, DIM), full),
    ]
    args = [x, agg1, agg2, u1x, u1g, ub1, u2, ub2, u3, ub3]
    nblk = pl.BlockSpec((NBLK, DIM), lambda i: (i, 0))
    hblk = pl.BlockSpec((NBLK, HALF), lambda i: (i, 0))
    nshape = jax.ShapeDtypeStruct((NP, DIM), _f32)
    hshape = jax.ShapeDtypeStruct((NP, HALF), _i32)
    if emit_pq:
        in_specs += [pl.BlockSpec((DIM, DIM), full), pl.BlockSpec((DIM, DIM), full)]
        args += [wa, wb]
        out_specs = [nblk, hblk, hblk]
        out_shape = [nshape, hshape, hshape]
    else:
        out_specs = [nblk]
        out_shape = [nshape]
    return pl.pallas_call(
        functools.partial(_tc_upd_body, emit_pq),
        grid=(NP // NBLK,),
        in_specs=in_specs,
        out_specs=out_specs,
        out_shape=out_shape,
    )(*args)


def _tc_tail_body(x_ref, batch_ref, h1_ref, hb1_ref, h2_ref, hb2_ref,
                  h3_ref, hb3_ref, out_ref, acc_s, cnt_s):
    i = pl.program_id(0)

    @pl.when(i == 0)
    def _init():
        acc_s[...] = jnp.zeros_like(acc_s)
        cnt_s[...] = jnp.zeros_like(cnt_s)

    gids = lax.broadcasted_iota(_i32, (1, N_GRAPHS), 1)
    oh = (batch_ref[...] == gids).astype(_f32)
    x = x_ref[...]
    acc_s[...] += lax.dot_general(oh, x, (((0,), (0,)), ((), ())),
                                  preferred_element_type=_f32)
    ones = jnp.ones((NBLK, 1), _f32)
    cnt_s[...] += lax.dot_general(oh, ones, (((0,), (0,)), ((), ())),
                                  preferred_element_type=_f32)

    @pl.when(i == pl.num_programs(0) - 1)
    def _final():
        pooled = acc_s[...] / jnp.maximum(cnt_s[...], 1.0)
        h = _silu(jnp.dot(pooled, h1_ref[...], preferred_element_type=_f32)
                  + hb1_ref[...])
        h = _silu(jnp.dot(h, h2_ref[...], preferred_element_type=_f32)
                  + hb2_ref[...])
        out_ref[...] = (jnp.dot(h, h3_ref[...], preferred_element_type=_f32)
                        + hb3_ref[...])


def _tc_tail(x, batch2, h1, hb1, h2, hb2, h3, hb3):
    full = lambda i: (0, 0)
    return pl.pallas_call(
        _tc_tail_body,
        grid=(NP // NBLK,),
        in_specs=[
            pl.BlockSpec((NBLK, DIM), lambda i: (i, 0)),
            pl.BlockSpec((NBLK, 1), lambda i: (i, 0)),
            pl.BlockSpec((DIM, DIM), full),
            pl.BlockSpec((1, DIM), full),
            pl.BlockSpec((DIM, DIM), full),
            pl.BlockSpec((1, DIM), full),
            pl.BlockSpec((DIM, 1), full),
            pl.BlockSpec((1, 1), full),
        ],
        out_specs=pl.BlockSpec((N_GRAPHS, 1), full),
        out_shape=jax.ShapeDtypeStruct((N_GRAPHS, 1), _f32),
        scratch_shapes=[
            pltpu.VMEM((N_GRAPHS, DIM), _f32),
            pltpu.VMEM((N_GRAPHS, 1), _f32),
        ],
    )(x, batch2, h1, hb1, h2, hb2, h3, hb3)


def kernel(z, edge_index, batch, pos, emb, freq, layers, head):
    row = edge_index[0].astype(_i32)
    col = edge_index[1].astype(_i32)
    rowp = jnp.pad(row, (0, EP - N_EDGES))
    colp = jnp.pad(col, (0, EP - N_EDGES))
    sidx = jnp.pad(col, (0, EP - N_EDGES), constant_values=N_NODES)
    zp = jnp.pad(z.astype(_i32), (0, NP - N_NODES))
    batchp = jnp.pad(batch.astype(_i32), (0, NP - N_NODES),
                     constant_values=N_GRAPHS).reshape(NP, 1)
    posT = pos.T.reshape(3 * N_NODES)
    freq2 = freq.reshape(1, NUM_BASIS)

    x0, dd = _sc_prep(posT, zp, rowp, colp, emb)
    dd2 = dd.reshape(EP, 1)

    def msg_parts(layer):
        (w1, b1), (w2, b2), (w3, b3) = layer["msg"]
        return (w1[:DIM], w1[DIM:2 * DIM], w1[2 * DIM:],
                b1.reshape(1, DIM), w2, b2.reshape(1, DIM), w3,
                b3.reshape(1, DIM))

    def upd_parts(layer):
        (u1, ub1), (u2, ub2), (u3, ub3) = layer["upd"]
        return (u1[:DIM], u1[DIM:], ub1.reshape(1, DIM), u2,
                ub2.reshape(1, DIM), u3, ub3.reshape(1, DIM))

    x = x0
    wa0, wb0 = msg_parts(layers[0])[0], msg_parts(layers[0])[1]
    p, q = _tc_pq(x0, wa0, wb0)
    for li, layer in enumerate(layers):
        _, _, w1c, b1, w2, b2, w3, b3 = msg_parts(layer)
        s1 = _sc_gather0(p, q, rowp, colp)
        m1 = _tc_edge(s1, dd2, freq2, w1c, b1, w2, b2, w3, b3, 0)
        s2 = _sc_gather1(p, q, rowp, colp)
        agg1 = _sc_scatter0(m1, sidx)
        m2 = _tc_edge(s2, dd2, freq2, w1c, b1, w2, b2, w3, b3, 1)
        agg2 = _sc_scatter1(m2, sidx)
        u1x, u1g, ub1, u2, ub2, u3, ub3 = upd_parts(layer)
        if li + 1 < len(layers):
            wa, wb = msg_parts(layers[li + 1])[0], msg_parts(layers[li + 1])[1]
            x, p, q = _tc_upd(x, agg1, agg2, u1x, u1g, ub1, u2, ub2, u3, ub3,
                              wa, wb)
        else:
            (x,) = _tc_upd(x, agg1, agg2, u1x, u1g, ub1, u2, ub2, u3, ub3)

    (h1, hb1), (h2, hb2), (h3, hb3) = head
    return _tc_tail(x, batchp, h1, hb1.reshape(1, DIM), h2,
                    hb2.reshape(1, DIM), h3, hb3.reshape(1, 1))


# speedup vs baseline: 1.2800x; 1.0181x over previous
import functools

import jax
import jax.numpy as jnp
from jax import lax
from jax.experimental import pallas as pl
from jax.experimental.pallas import tpu as pltpu
from jax.experimental.pallas import tpu_sc as plsc

N_NODES = 10000
N_EDGES = 160000
DIM = 256
HALF = 128
NUM_BASIS = 16
CUTOFF = 5.0
N_GRAPHS = 8

NC = 2
NS = 16
NW = NC * NS
NP = 10240
EP = 163840
CHUNK = 128
E_PER_W = EP // NW
N_CHUNKS = E_PER_W // CHUNK
N_PER_W = NP // NW
ACC_ROWS = NP
EHALF = EP // 2

_f32 = jnp.float32
_i32 = jnp.int32
_bf16 = jnp.bfloat16


def _silu(x):
    return x * jax.nn.sigmoid(x)


_SC_MESH = plsc.VectorSubcoreMesh(core_axis_name="c", subcore_axis_name="s")


def _sc_prep_body(posT_hbm, z_hbm, row_hbm, col_hbm, emb_hbm,
                  x0_hbm, dd_hbm,
                  posT_v, zidx_v, ridx_v, cidx_v, dd_v, rows_v, sem):
    c = lax.axis_index("c")
    s = lax.axis_index("s")
    w = c * NS + s

    pltpu.sync_copy(posT_hbm, posT_v)

    for j in range(5):
        base = w * N_PER_W + j * 64
        pltpu.sync_copy(z_hbm.at[pl.ds(base, 64)], zidx_v)
        pltpu.async_copy(emb_hbm.at[zidx_v], rows_v, sem).wait()
        pltpu.sync_copy(rows_v, x0_hbm.at[pl.ds(base, 64)])

    def dd_chunk(j, carry):
        base = w * E_PER_W + j * CHUNK
        pltpu.sync_copy(row_hbm.at[pl.ds(base, CHUNK)], ridx_v)
        pltpu.sync_copy(col_hbm.at[pl.ds(base, CHUNK)], cidx_v)
        for g in range(CHUNK // 16):
            r = ridx_v[pl.ds(g * 16, 16)]
            cc = cidx_v[pl.ds(g * 16, 16)]
            acc = jnp.zeros((16,), _f32)
            for d in range(3):
                off = jnp.full((16,), d * N_NODES, _i32)
                xr = plsc.load_gather(posT_v, [off + r])
                xc = plsc.load_gather(posT_v, [off + cc])
                df = xr - xc
                acc = acc + df * df
            dd_v[pl.ds(g * 16, 16)] = acc
        pltpu.sync_copy(dd_v, dd_hbm.at[pl.ds(base, CHUNK)])
        return carry

    lax.fori_loop(0, N_CHUNKS, dd_chunk, 0)


_sc_prep = pl.kernel(
    _sc_prep_body,
    out_type=[
        jax.ShapeDtypeStruct((NP, DIM), _f32),
        jax.ShapeDtypeStruct((EP,), _f32),
    ],
    mesh=_SC_MESH,
    scratch_types=[
        pltpu.VMEM((3 * N_NODES,), _f32),
        pltpu.VMEM((64,), _i32),
        pltpu.VMEM((CHUNK,), _i32),
        pltpu.VMEM((CHUNK,), _i32),
        pltpu.VMEM((CHUNK,), _f32),
        pltpu.VMEM((64, DIM), _f32),
        pltpu.SemaphoreType.DMA,
    ],
    compiler_params=pltpu.CompilerParams(needs_layout_passes=False),
)


EH_PER_W = EHALF // NW
NH_CHUNKS = EH_PER_W // CHUNK


def _sc_gather_body(off, p_hbm, q_hbm, row_hbm, col_hbm,
                    s_hbm,
                    cidx0, ridx0, bp0, bq0,
                    cidx1, ridx1, bp1, bq1,
                    semg0, semg1, semw0, semw1):
    c = lax.axis_index("c")
    s = lax.axis_index("s")
    w = c * NS + s
    wbase = w * EH_PER_W

    slots = ((cidx0, ridx0, bp0, bq0, semg0, semw0),
             (cidx1, ridx1, bp1, bq1, semg1, semw1))

    def start(j, b):
        cidx, ridx, bp, bq, semg, _ = slots[b]
        base = wbase + j * CHUNK
        pltpu.sync_copy(col_hbm.at[pl.ds(off + base, CHUNK)], cidx)
        pltpu.sync_copy(row_hbm.at[pl.ds(off + base, CHUNK)], ridx)
        pltpu.async_copy(p_hbm.at[cidx], bp, semg)
        pltpu.async_copy(q_hbm.at[ridx], bq, semg)

    def finish(j, b):
        cidx, ridx, bp, bq, semg, semw = slots[b]
        base = wbase + j * CHUNK
        pltpu.make_async_copy(p_hbm.at[cidx], bp, semg).wait()
        pltpu.make_async_copy(q_hbm.at[ridx], bq, semg).wait()

        def add_row(r, carry2):
            for g in range(HALF // 16):
                sl = pl.ds(g * 16, 16)
                a = plsc.bitcast(bp[r, sl], _bf16)
                b = plsc.bitcast(bq[r, sl], _bf16)
                bp[r, sl] = plsc.bitcast(a + b, _i32)
            return carry2

        lax.fori_loop(0, CHUNK, add_row, 0)
        pltpu.async_copy(bp, s_hbm.at[pl.ds(base, CHUNK)], semw)

    def wait_wb(j, b):
        _, _, bp, _, _, semw = slots[b]
        base = wbase + j * CHUNK
        pltpu.make_async_copy(bp, s_hbm.at[pl.ds(base, CHUNK)], semw).wait()

    start(0, 0)
    start(1, 1)

    def pair(jj, carry):
        j0 = 2 * jj
        finish(j0, 0)
        wait_wb(j0, 0)
        start(j0 + 2, 0)
        finish(j0 + 1, 1)
        wait_wb(j0 + 1, 1)
        start(j0 + 3, 1)
        return carry

    lax.fori_loop(0, NH_CHUNKS // 2 - 1, pair, 0)
    finish(NH_CHUNKS - 2, 0)
    finish(NH_CHUNKS - 1, 1)
    wait_wb(NH_CHUNKS - 2, 0)
    wait_wb(NH_CHUNKS - 1, 1)


def _make_gather(off):
    return pl.kernel(
        functools.partial(_sc_gather_body, off),
        out_type=jax.ShapeDtypeStruct((EHALF, HALF), _i32),
        mesh=_SC_MESH,
        scratch_types=[
            pltpu.VMEM((CHUNK,), _i32),
            pltpu.VMEM((CHUNK,), _i32),
            pltpu.VMEM((CHUNK, HALF), _i32),
            pltpu.VMEM((CHUNK, HALF), _i32),
            pltpu.VMEM((CHUNK,), _i32),
            pltpu.VMEM((CHUNK,), _i32),
            pltpu.VMEM((CHUNK, HALF), _i32),
            pltpu.VMEM((CHUNK, HALF), _i32),
            pltpu.SemaphoreType.DMA,
            pltpu.SemaphoreType.DMA,
            pltpu.SemaphoreType.DMA,
            pltpu.SemaphoreType.DMA,
        ],
        compiler_params=pltpu.CompilerParams(needs_layout_passes=False),
    )


_sc_gather0 = _make_gather(0)
_sc_gather1 = _make_gather(EHALF)


def _sc_scatter_body(off, m_hbm, sidx_hbm,
                     agg_hbm,
                     idx0, mb0, idx1, mb1, zbuf, acc_sh, semm0, semm1):
    c = lax.axis_index("c")
    s = lax.axis_index("s")

    def zrow(r, carry):
        for g in range(HALF // 16):
            zbuf[r, pl.ds(g * 16, 16)] = jnp.zeros((16,), _f32)
        return carry

    lax.fori_loop(0, 64, zrow, 0)
    rows_per_sub = ACC_ROWS // NS
    for k in range(rows_per_sub // 64):
        pltpu.sync_copy(zbuf, acc_sh.at[pl.ds(s * rows_per_sub + k * 64, 64)])
    plsc.subcore_barrier()

    e_per_sub = EHALF // NS
    n_chunks = e_per_sub // CHUNK
    sbase = s * e_per_sub

    slots = ((idx0, mb0, semm0), (idx1, mb1, semm1))

    def startm(j, b):
        idx, mb, semm = slots[b]
        base = sbase + j * CHUNK
        pltpu.sync_copy(sidx_hbm.at[pl.ds(off + base, CHUNK)], idx)
        pltpu.async_copy(m_hbm.at[c, pl.ds(base, CHUNK)], mb, semm)

    def finishm(j, b):
        idx, mb, semm = slots[b]
        base = sbase + j * CHUNK
        pltpu.make_async_copy(m_hbm.at[c, pl.ds(base, CHUNK)], mb, semm).wait()
        pltpu.sync_copy(mb, acc_sh.at[idx], add=True)

    startm(0, 0)
    startm(1, 1)

    def pair(jj, carry):
        j0 = 2 * jj
        finishm(j0, 0)
        startm(j0 + 2, 0)
        finishm(j0 + 1, 1)
        startm(j0 + 3, 1)
        return carry

    lax.fori_loop(0, n_chunks // 2 - 1, pair, 0)
    finishm(n_chunks - 2, 0)
    finishm(n_chunks - 1, 1)
    plsc.subcore_barrier()

    out_per_sub = NP // NS
    pltpu.sync_copy(acc_sh.at[pl.ds(s * out_per_sub, out_per_sub)],
                    agg_hbm.at[c, pl.ds(s * out_per_sub, out_per_sub)])


def _make_scatter(off):
    return pl.kernel(
        functools.partial(_sc_scatter_body, off),
        out_type=jax.ShapeDtypeStruct((NC, NP, HALF), _f32),
        mesh=_SC_MESH,
        scratch_types=[
            pltpu.VMEM((CHUNK,), _i32),
            pltpu.VMEM((CHUNK, HALF), _f32),
            pltpu.VMEM((CHUNK,), _i32),
            pltpu.VMEM((CHUNK, HALF), _f32),
            pltpu.VMEM((64, HALF), _f32),
            pltpu.VMEM_SHARED((ACC_ROWS, HALF), _f32),
            pltpu.SemaphoreType.DMA,
            pltpu.SemaphoreType.DMA,
        ],
    )


_sc_scatter0 = _make_scatter(0)
_sc_scatter1 = _make_scatter(EHALF)


EBLK = 4096
NBLK = 1024


def _pack_tc(a):
    lo = lax.bitcast_convert_type(
        a[:, :HALF].astype(_bf16).astype(_f32), _i32)
    hi = lax.bitcast_convert_type(
        a[:, HALF:].astype(_bf16).astype(_f32), _i32)
    return jnp.bitwise_or(lax.shift_right_logical(lo, jnp.int32(16)),
                          jnp.bitwise_and(hi, jnp.int32(-65536)))


def _unpack_tc(w):
    lo = lax.bitcast_convert_type(lax.shift_left(w, jnp.int32(16)), _f32)
    hi = lax.bitcast_convert_type(jnp.bitwise_and(w, jnp.int32(-65536)), _f32)
    return jnp.concatenate([lo, hi], axis=1)


def _tc_pq_body(x_ref, wa_ref, wb_ref, p_ref, q_ref):
    x = x_ref[...]
    p_ref[...] = _pack_tc(jnp.dot(x, wa_ref[...], preferred_element_type=_f32))
    q_ref[...] = _pack_tc(jnp.dot(x, wb_ref[...], preferred_element_type=_f32))


def _tc_pq(x, wa, wb):
    return pl.pallas_call(
        _tc_pq_body,
        grid=(NP // NBLK,),
        in_specs=[
            pl.BlockSpec((NBLK, DIM), lambda i: (i, 0)),
            pl.BlockSpec((DIM, DIM), lambda i: (0, 0)),
            pl.BlockSpec((DIM, DIM), lambda i: (0, 0)),
        ],
        out_specs=[
            pl.BlockSpec((NBLK, HALF), lambda i: (i, 0)),
            pl.BlockSpec((NBLK, HALF), lambda i: (i, 0)),
        ],
        out_shape=[
            jax.ShapeDtypeStruct((NP, HALF), _i32),
            jax.ShapeDtypeStruct((NP, HALF), _i32),
        ],
    )(x, wa, wb)


def _bessel_block(dd, freq):
    dist = jnp.sqrt(dd + 1e-12)
    ds = dist / CUTOFF
    p = 6.0
    a = -(p + 1.0) * (p + 2.0) / 2.0
    b = p * (p + 2.0)
    c = -p * (p + 1.0) / 2.0
    ds2 = ds * ds
    ds4 = ds2 * ds2
    ds5 = ds4 * ds
    ds6 = ds5 * ds
    ds7 = ds6 * ds
    env = (1.0 / ds + a * ds5 + b * ds6 + c * ds7) * (ds < 1.0).astype(_f32)
    return env * jnp.sin(freq * ds)


def _tc_edge_body(s_ref, dd_ref, freq_ref, w1c_ref, b1_ref,
                  w2_ref, b2_ref, w3_ref, b3_ref, m_ref):
    ea = _bessel_block(dd_ref[...], freq_ref[...])
    h = _unpack_tc(s_ref[...]) + lax.dot_general(
        ea, w1c_ref[...], (((1,), (0,)), ((), ())),
        preferred_element_type=_f32) + b1_ref[...]
    h = _silu(h)
    h = _silu(jnp.dot(h, w2_ref[...], preferred_element_type=_f32) + b2_ref[...])
    m = jnp.dot(h, w3_ref[...], preferred_element_type=_f32) + b3_ref[...]
    m_ref[0] = m[:, :HALF]
    m_ref[1] = m[:, HALF:]


def _tc_edge(s, dd2, freq2, w1c, b1, w2, b2, w3, b3, half):
    nblk = EHALF // EBLK
    off = half * nblk
    return pl.pallas_call(
        _tc_edge_body,
        grid=(nblk,),
        in_specs=[
            pl.BlockSpec((EBLK, HALF), lambda i: (i, 0)),
            pl.BlockSpec((EBLK, 1), lambda i: (i + off, 0)),
            pl.BlockSpec((1, NUM_BASIS), lambda i: (0, 0)),
            pl.BlockSpec((NUM_BASIS, DIM), lambda i: (0, 0)),
            pl.BlockSpec((1, DIM), lambda i: (0, 0)),
            pl.BlockSpec((DIM, DIM), lambda i: (0, 0)),
            pl.BlockSpec((1, DIM), lambda i: (0, 0)),
            pl.BlockSpec((DIM, DIM), lambda i: (0, 0)),
            pl.BlockSpec((1, DIM), lambda i: (0, 0)),
        ],
        out_specs=pl.BlockSpec((NC, EBLK, HALF), lambda i: (0, i, 0)),
        out_shape=jax.ShapeDtypeStruct((NC, EHALF, HALF), _f32),
    )(s, dd2, freq2, w1c, b1, w2, b2, w3, b3)


def _tc_upd_body(emit_pq, x_ref, agg1_ref, agg2_ref, u1x_ref, u1g_ref,
                 ub1_ref, u2_ref, ub2_ref, u3_ref, ub3_ref, *rest):
    if emit_pq:
        wa_ref, wb_ref, xn_ref, p_ref, q_ref = rest
    else:
        (xn_ref,) = rest
    x = x_ref[...]
    aggc = jnp.concatenate([agg1_ref[0] + agg2_ref[0],
                            agg1_ref[1] + agg2_ref[1]], axis=1)
    h = (jnp.dot(x, u1x_ref[...], preferred_element_type=_f32)
         + jnp.dot(aggc, u1g_ref[...], preferred_element_type=_f32)
         + ub1_ref[...])
    h = _silu(h)
    h = _silu(jnp.dot(h, u2_ref[...], preferred_element_type=_f32) + ub2_ref[...])
    xn = x + jnp.dot(h, u3_ref[...], preferred_element_type=_f32) + ub3_ref[...]
    xn_ref[...] = xn
    if emit_pq:
        p_ref[...] = _pack_tc(jnp.dot(xn, wa_ref[...],
                                      preferred_element_type=_f32))
        q_ref[...] = _pack_tc(jnp.dot(xn, wb_ref[...],
                                      preferred_element_type=_f32))


def _tc_upd(x, agg1, agg2, u1x, u1g, ub1, u2, ub2, u3, ub3, wa=None, wb=None):
    emit_pq = wa is not None
    full = lambda i: (0, 0)
    in_specs = [
        pl.BlockSpec((NBLK, DIM), lambda i: (i, 0)),
        pl.BlockSpec((NC, NBLK, HALF), lambda i: (0, i, 0)),
        pl.BlockSpec((NC, NBLK, HALF), lambda i: (0, i, 0)),
        pl.BlockSpec((DIM, DIM), full),
        pl.BlockSpec((DIM, DIM), full),
        pl.BlockSpec((1, DIM), full),
        pl.BlockSpec((DIM, DIM), full),
        pl.BlockSpec((1, DIM), full),
        pl.BlockSpec((DIM, DIM), full),
        pl.BlockSpec((1, DIM), full),
    ]
    args = [x, agg1, agg2, u1x, u1g, ub1, u2, ub2, u3, ub3]
    nblk = pl.BlockSpec((NBLK, DIM), lambda i: (i, 0))
    hblk = pl.BlockSpec((NBLK, HALF), lambda i: (i, 0))
    nshape = jax.ShapeDtypeStruct((NP, DIM), _f32)
    hshape = jax.ShapeDtypeStruct((NP, HALF), _i32)
    if emit_pq:
        in_specs += [pl.BlockSpec((DIM, DIM), full), pl.BlockSpec((DIM, DIM), full)]
        args += [wa, wb]
        out_specs = [nblk, hblk, hblk]
        out_shape = [nshape, hshape, hshape]
    else:
        out_specs = [nblk]
        out_shape = [nshape]
    return pl.pallas_call(
        functools.partial(_tc_upd_body, emit_pq),
        grid=(NP // NBLK,),
        in_specs=in_specs,
        out_specs=out_specs,
        out_shape=out_shape,
    )(*args)


def _tc_tail_body(x_ref, batch_ref, h1_ref, hb1_ref, h2_ref, hb2_ref,
                  h3_ref, hb3_ref, out_ref, acc_s, cnt_s):
    i = pl.program_id(0)

    @pl.when(i == 0)
    def _init():
        acc_s[...] = jnp.zeros_like(acc_s)
        cnt_s[...] = jnp.zeros_like(cnt_s)

    gids = lax.broadcasted_iota(_i32, (1, N_GRAPHS), 1)
    oh = (batch_ref[...] == gids).astype(_f32)
    x = x_ref[...]
    acc_s[...] += lax.dot_general(oh, x, (((0,), (0,)), ((), ())),
                                  preferred_element_type=_f32)
    ones = jnp.ones((NBLK, 1), _f32)
    cnt_s[...] += lax.dot_general(oh, ones, (((0,), (0,)), ((), ())),
                                  preferred_element_type=_f32)

    @pl.when(i == pl.num_programs(0) - 1)
    def _final():
        pooled = acc_s[...] / jnp.maximum(cnt_s[...], 1.0)
        h = _silu(jnp.dot(pooled, h1_ref[...], preferred_element_type=_f32)
                  + hb1_ref[...])
        h = _silu(jnp.dot(h, h2_ref[...], preferred_element_type=_f32)
                  + hb2_ref[...])
        out_ref[...] = (jnp.dot(h, h3_ref[...], preferred_element_type=_f32)
                        + hb3_ref[...])


def _tc_tail(x, batch2, h1, hb1, h2, hb2, h3, hb3):
    full = lambda i: (0, 0)
    return pl.pallas_call(
        _tc_tail_body,
        grid=(NP // NBLK,),
        in_specs=[
            pl.BlockSpec((NBLK, DIM), lambda i: (i, 0)),
            pl.BlockSpec((NBLK, 1), lambda i: (i, 0)),
            pl.BlockSpec((DIM, DIM), full),
            pl.BlockSpec((1, DIM), full),
            pl.BlockSpec((DIM, DIM), full),
            pl.BlockSpec((1, DIM), full),
            pl.BlockSpec((DIM, 1), full),
            pl.BlockSpec((1, 1), full),
        ],
        out_specs=pl.BlockSpec((N_GRAPHS, 1), full),
        out_shape=jax.ShapeDtypeStruct((N_GRAPHS, 1), _f32),
        scratch_shapes=[
            pltpu.VMEM((N_GRAPHS, DIM), _f32),
            pltpu.VMEM((N_GRAPHS, 1), _f32),
        ],
    )(x, batch2, h1, hb1, h2, hb2, h3, hb3)


def kernel(z, edge_index, batch, pos, emb, freq, layers, head):
    row = edge_index[0].astype(_i32)
    col = edge_index[1].astype(_i32)
    rowp = jnp.pad(row, (0, EP - N_EDGES))
    colp = jnp.pad(col, (0, EP - N_EDGES))
    sidx = jnp.pad(col, (0, EP - N_EDGES), constant_values=N_NODES)
    zp = jnp.pad(z.astype(_i32), (0, NP - N_NODES))
    batchp = jnp.pad(batch.astype(_i32), (0, NP - N_NODES),
                     constant_values=N_GRAPHS).reshape(NP, 1)
    posT = pos.T.reshape(3 * N_NODES)
    freq2 = freq.reshape(1, NUM_BASIS)

    x0, dd = _sc_prep(posT, zp, rowp, colp, emb)
    dd2 = dd.reshape(EP, 1)

    def msg_parts(layer):
        (w1, b1), (w2, b2), (w3, b3) = layer["msg"]
        return (w1[:DIM], w1[DIM:2 * DIM], w1[2 * DIM:],
                b1.reshape(1, DIM), w2, b2.reshape(1, DIM), w3,
                b3.reshape(1, DIM))

    def upd_parts(layer):
        (u1, ub1), (u2, ub2), (u3, ub3) = layer["upd"]
        return (u1[:DIM], u1[DIM:], ub1.reshape(1, DIM), u2,
                ub2.reshape(1, DIM), u3, ub3.reshape(1, DIM))

    x = x0
    wa0, wb0 = msg_parts(layers[0])[0], msg_parts(layers[0])[1]
    p, q = _tc_pq(x0, wa0, wb0)
    for li, layer in enumerate(layers):
        _, _, w1c, b1, w2, b2, w3, b3 = msg_parts(layer)
        s1 = _sc_gather0(p, q, rowp, colp)
        m1 = _tc_edge(s1, dd2, freq2, w1c, b1, w2, b2, w3, b3, 0)
        s2 = _sc_gather1(p, q, rowp, colp)
        agg1 = _sc_scatter0(m1, sidx)
        m2 = _tc_edge(s2, dd2, freq2, w1c, b1, w2, b2, w3, b3, 1)
        agg2 = _sc_scatter1(m2, sidx)
        u1x, u1g, ub1, u2, ub2, u3, ub3 = upd_parts(layer)
        if li + 1 < len(layers):
            wa, wb = msg_parts(layers[li + 1])[0], msg_parts(layers[li + 1])[1]
            x, p, q = _tc_upd(x, agg1, agg2, u1x, u1g, ub1, u2, ub2, u3, ub3,
                              wa, wb)
        else:
            (x,) = _tc_upd(x, agg1, agg2, u1x, u1g, ub1, u2, ub2, u3, ub3)

    (h1, hb1), (h2, hb2), (h3, hb3) = head
    return _tc_tail(x, batchp, h1, hb1.reshape(1, DIM), h2,
                    hb2.reshape(1, DIM), h3, hb3.reshape(1, 1))
